# Initial kernel scaffold; baseline (speedup 1.0000x reference)
import jax
import jax.numpy as jnp
from jax.experimental import pallas as pl


def kernel(x, edge_index, edge_weight, W1, b1, W2, b2, Wl, bl):
    raise NotImplementedError("write your pallas kernel here")


# speedup vs baseline: 8.3209x; 8.3209x over previous
import functools

import jax
import jax.numpy as jnp
from jax import lax
from jax.experimental import pallas as pl
from jax.experimental.pallas import tpu as pltpu
from jax.experimental.pallas import tpu_sc as plsc

NC, NS, L = 2, 16, 16


def _mesh():
    return plsc.VectorSubcoreMesh(
        core_axis_name="c", subcore_axis_name="s", num_cores=NC, num_subcores=NS
    )


def _make_deg(E, N):
    CH = 128
    n_chunks = E // CH
    NW = NC * NS
    cpw = -(-n_chunks // NW)
    span = ((-(-N // NS)) + 127) // 128 * 128
    NP = span * NS

    @functools.partial(
        pl.kernel,
        out_type=jax.ShapeDtypeStruct((NC * NP,), jnp.float32),
        mesh=_mesh(),
        scratch_types=[
            pltpu.VMEM((CH,), jnp.int32),
            pltpu.VMEM((CH,), jnp.float32),
            pltpu.VMEM((NP,), jnp.float32),
            pltpu.VMEM((span,), jnp.float32),
            pltpu.VMEM((span,), jnp.float32),
            pltpu.VMEM_SHARED((NS * NP,), jnp.float32),
        ],
        compiler_params=pltpu.CompilerParams(needs_layout_passes=False),
    )
    def deg_kernel(col_hbm, ew_hbm, out_hbm, col_v, ew_v, deg_v, acc_v, tmp_v, shared):
        cid = lax.axis_index("c")
        sid = lax.axis_index("s")
        w = sid * NC + cid
        zero16 = jnp.zeros((L,), jnp.float32)

        def zbody(i, c):
            deg_v[pl.ds(i * L, L)] = zero16
            return c
        lax.fori_loop(0, NP // L, zbody, 0)

        def ebody(it, c):
            chunk = w + it * NW

            @pl.when(chunk < n_chunks)
            def _():
                base = chunk * CH
                pltpu.sync_copy(col_hbm.at[pl.ds(base, CH)], col_v)
                pltpu.sync_copy(ew_hbm.at[pl.ds(base, CH)], ew_v)
                for g in range(CH // L):
                    cvec = col_v[pl.ds(g * L, L)]
                    wvec = ew_v[pl.ds(g * L, L)]
                    plsc.addupdate_scatter(deg_v, [cvec], wvec)
            return c
        lax.fori_loop(0, cpw, ebody, 0)

        pltpu.sync_copy(deg_v, shared.at[pl.ds(sid * NP, NP)])
        plsc.subcore_barrier()

        off = sid * span

        def z2(i, c):
            acc_v[pl.ds(i * L, L)] = zero16
            return c
        lax.fori_loop(0, span // L, z2, 0)
        for t in range(NS):
            pltpu.sync_copy(shared.at[pl.ds(t * NP + off, span)], tmp_v)

            def rbody(i, c):
                sl = pl.ds(i * L, L)
                acc_v[sl] = acc_v[sl] + tmp_v[sl]
                return c
            lax.fori_loop(0, span // L, rbody, 0)
        pltpu.sync_copy(acc_v, out_hbm.at[pl.ds(cid * NP + off, span)])

    return deg_kernel, NP


def _make_edge(E, N, D):
    CH = 128
    n_chunks = E // CH
    cpt = -(-n_chunks // NS)
    nfull = N // 128
    rem = N - nfull * 128
    nhop = -(-(nfull + (1 if rem else 0)) // NS)

    @functools.partial(
        pl.kernel,
        out_type=jax.ShapeDtypeStruct((NC * N, D), jnp.float32),
        mesh=_mesh(),
        scratch_types=[
            pltpu.VMEM((CH,), jnp.int32),
            pltpu.VMEM((CH,), jnp.int32),
            pltpu.VMEM((CH,), jnp.float32),
            pltpu.VMEM((CH, 128), jnp.float32),
            pltpu.VMEM_SHARED((N, 128), jnp.float32),
            pltpu.SemaphoreType.DMA,
        ],
    )
    def edge_kernel(y_hbm, row_hbm, col_hbm, ew_hbm, out_hbm,
                    idx_v, col_v, ew_v, rows_v, acc_sh, sem):
        cid = lax.axis_index("c")
        sid = lax.axis_index("s")

        def ibody(ih, c):
            h = sid + ih * NS

            @pl.when(h < nfull)
            def _():
                r0 = h * 128
                pltpu.sync_copy(y_hbm.at[pl.ds(cid * N + r0, 128)], rows_v)
                pltpu.sync_copy(rows_v, acc_sh.at[pl.ds(r0, 128)])
            if rem:
                @pl.when(h == nfull)
                def _():
                    r0 = nfull * 128
                    pltpu.sync_copy(y_hbm.at[pl.ds(cid * N + r0, rem)],
                                    rows_v.at[pl.ds(0, rem)])
                    pltpu.sync_copy(rows_v.at[pl.ds(0, rem)],
                                    acc_sh.at[pl.ds(r0, rem)])
            return c
        lax.fori_loop(0, nhop, ibody, 0)
        plsc.subcore_barrier()

        def ebody(it, c):
            chunk = sid + it * NS

            @pl.when(chunk < n_chunks)
            def _():
                base = chunk * CH
                pltpu.sync_copy(row_hbm.at[pl.ds(base, CH)], idx_v)
                pltpu.sync_copy(col_hbm.at[pl.ds(base, CH)], col_v)
                pltpu.sync_copy(ew_hbm.at[pl.ds(base, CH)], ew_v)
                offv = cid * N
                for g in range(CH // L):
                    sl = pl.ds(g * L, L)
                    idx_v[sl] = idx_v[sl] + offv
                pltpu.async_copy(y_hbm.at[idx_v], rows_v, sem).wait()

                def sbody(jg, c2):
                    wv = ew_v[pl.ds(jg * L, L)]
                    for l in range(L):
                        s = wv[l]
                        j = jg * L + l
                        for k in range(128 // L):
                            sl = pl.ds(k * L, L)
                            rows_v[j, sl] = rows_v[j, sl] * s
                    return c2
                lax.fori_loop(0, CH // L, sbody, 0)
                pltpu.sync_copy(rows_v, acc_sh.at[col_v], add=True)
            return c
        lax.fori_loop(0, cpt, ebody, 0)

        plsc.subcore_barrier()

        def obody(ih, c):
            h = sid + ih * NS

            @pl.when(h < nfull)
            def _():
                r0 = h * 128
                pltpu.sync_copy(acc_sh.at[pl.ds(r0, 128)], rows_v)
                pltpu.sync_copy(rows_v, out_hbm.at[pl.ds(cid * N + r0, 128)])
            if rem:
                @pl.when(h == nfull)
                def _():
                    r0 = nfull * 128
                    pltpu.sync_copy(acc_sh.at[pl.ds(r0, rem)],
                                    rows_v.at[pl.ds(0, rem)])
                    pltpu.sync_copy(rows_v.at[pl.ds(0, rem)],
                                    out_hbm.at[pl.ds(cid * N + r0, rem)])
            return c
        lax.fori_loop(0, nhop, obody, 0)

    return edge_kernel


def _tc1_body(x_ref, w_ref, dis_ref, y_ref):
    xw = jnp.dot(x_ref[...], w_ref[...], preferred_element_type=jnp.float32)
    y = xw * dis_ref[...]
    h = xw.shape[1] // 2
    y_ref[0] = y[:, :h]
    y_ref[1] = y[:, h:]


def _tc2_body(a_ref, dis_ref, b_ref, w_ref, y_ref):
    d = dis_ref[...]
    hcat = jnp.concatenate([a_ref[0], a_ref[1]], axis=1)
    hid = jnp.maximum(hcat * d + b_ref[...], 0.0)
    y = jnp.dot(hid, w_ref[...], preferred_element_type=jnp.float32) * d
    h = y.shape[1] // 2
    y_ref[0] = y[:, :h]
    y_ref[1] = y[:, h:]


def _tc3_body(a_ref, dis_ref, b_ref, w_ref, bl_ref, o_ref):
    d = dis_ref[...]
    hcat = jnp.concatenate([a_ref[0], a_ref[1]], axis=1)
    hid = jnp.maximum(hcat * d + b_ref[...], 0.0)
    o_ref[...] = (
        jnp.dot(hid, w_ref[...], preferred_element_type=jnp.float32) + bl_ref[...]
    )


def _tc1(x, W, dis_col, BR):
    N, K = x.shape
    H = W.shape[1]
    grid = (N // BR,)
    return pl.pallas_call(
        _tc1_body,
        grid=grid,
        in_specs=[
            pl.BlockSpec((BR, K), lambda i: (i, 0)),
            pl.BlockSpec((K, H), lambda i: (0, 0)),
            pl.BlockSpec((BR, 1), lambda i: (i, 0)),
        ],
        out_specs=pl.BlockSpec((2, BR, H // 2), lambda i: (0, i, 0)),
        out_shape=jax.ShapeDtypeStruct((2, N, H // 2), jnp.float32),
    )(x, W, dis_col)


def _tc2(acc, dis_col, b, W, BR):
    _, N, Hh = acc.shape
    H = W.shape[1]
    grid = (N // BR,)
    return pl.pallas_call(
        _tc2_body,
        grid=grid,
        in_specs=[
            pl.BlockSpec((2, BR, Hh), lambda i: (0, i, 0)),
            pl.BlockSpec((BR, 1), lambda i: (i, 0)),
            pl.BlockSpec((1, 2 * Hh), lambda i: (0, 0)),
            pl.BlockSpec((2 * Hh, H), lambda i: (0, 0)),
        ],
        out_specs=pl.BlockSpec((2, BR, H // 2), lambda i: (0, i, 0)),
        out_shape=jax.ShapeDtypeStruct((2, N, H // 2), jnp.float32),
    )(acc, dis_col, b, W)


def _tc3(acc, dis_col, b, W, bl, BR):
    _, N, Hh = acc.shape
    DO = W.shape[1]
    grid = (N // BR,)
    return pl.pallas_call(
        _tc3_body,
        grid=grid,
        in_specs=[
            pl.BlockSpec((2, BR, Hh), lambda i: (0, i, 0)),
            pl.BlockSpec((BR, 1), lambda i: (i, 0)),
            pl.BlockSpec((1, 2 * Hh), lambda i: (0, 0)),
            pl.BlockSpec((2 * Hh, DO), lambda i: (0, 0)),
            pl.BlockSpec((1, DO), lambda i: (0, 0)),
        ],
        out_specs=pl.BlockSpec((BR, DO), lambda i: (i, 0)),
        out_shape=jax.ShapeDtypeStruct((N, DO), jnp.float32),
    )(acc, dis_col, b, W, bl)


def kernel(x, edge_index, edge_weight, W1, b1, W2, b2, Wl, bl):
    N = x.shape[0]
    E = edge_weight.shape[0]
    BR = 1000

    row = edge_index[0]
    col = edge_index[1]

    deg_fn, NP = _make_deg(E, N)
    degp = deg_fn(col, edge_weight).reshape(2, NP)
    deg = degp[0, :N] + degp[1, :N] + 1.0
    dis = jnp.where(deg > 0, lax.rsqrt(deg), 0.0)
    dis_col = dis.reshape(N, 1)

    edge_fn = _make_edge(E, N, 128)

    y1 = _tc1(x, W1, dis_col, BR)
    a1 = edge_fn(y1.reshape(2 * N, 128), row, col, edge_weight)
    y2 = _tc2(a1.reshape(2, N, 128), dis_col, b1.reshape(1, -1), W2, BR)
    a2 = edge_fn(y2.reshape(2 * N, 128), row, col, edge_weight)
    out = _tc3(a2.reshape(2, N, 128), dis_col, b2.reshape(1, -1), Wl,
               bl.reshape(1, -1), BR)
    return out


# speedup vs baseline: 8.8710x; 1.0661x over previous
import functools

import jax
import jax.numpy as jnp
from jax import lax
from jax.experimental import pallas as pl
from jax.experimental.pallas import tpu as pltpu
from jax.experimental.pallas import tpu_sc as plsc

NC, NS, L = 2, 16, 16


def _mesh():
    return plsc.VectorSubcoreMesh(
        core_axis_name="c", subcore_axis_name="s", num_cores=NC, num_subcores=NS
    )


def _make_deg(E, N):
    CH = 128
    n_chunks = E // CH
    NW = NC * NS
    cpw = -(-n_chunks // NW)
    NP = (-(-N // 1024)) * 1024

    @functools.partial(
        pl.kernel,
        out_type=jax.ShapeDtypeStruct((NW * NP,), jnp.float32),
        mesh=_mesh(),
        scratch_types=[
            pltpu.VMEM((CH,), jnp.int32),
            pltpu.VMEM((CH,), jnp.float32),
            pltpu.VMEM((NP,), jnp.float32),
        ],
        compiler_params=pltpu.CompilerParams(needs_layout_passes=False),
    )
    def deg_kernel(col_hbm, ew_hbm, out_hbm, col_v, ew_v, deg_v):
        cid = lax.axis_index("c")
        sid = lax.axis_index("s")
        w = sid * NC + cid
        zero16 = jnp.zeros((L,), jnp.float32)

        def zbody(i, c):
            deg_v[pl.ds(i * L, L)] = zero16
            return c
        lax.fori_loop(0, NP // L, zbody, 0)

        def ebody(it, c):
            chunk = w + it * NW

            @pl.when(chunk < n_chunks)
            def _():
                base = chunk * CH
                pltpu.sync_copy(col_hbm.at[pl.ds(base, CH)], col_v)
                pltpu.sync_copy(ew_hbm.at[pl.ds(base, CH)], ew_v)
                for g in range(CH // L):
                    cvec = col_v[pl.ds(g * L, L)]
                    wvec = ew_v[pl.ds(g * L, L)]
                    plsc.addupdate_scatter(deg_v, [cvec], wvec)
            return c
        lax.fori_loop(0, cpw, ebody, 0)

        pltpu.sync_copy(deg_v, out_hbm.at[pl.ds(w * NP, NP)])

    return deg_kernel, NP


def _make_edge(E, N, D):
    CH = 128
    NB = 4
    AH = 2
    TPE = E // NS
    NCH = TPE // CH
    REM = TPE - NCH * CH
    NPASS = 2
    HN = ((-(-N // NPASS)) + 7) // 8 * 8
    LASTN = N - (NPASS - 1) * HN
    nfull = HN // 128
    rem = HN - nfull * 128
    nfull_l = LASTN // 128
    rem_l = LASTN - nfull_l * 128
    assert rem % 8 == 0 and rem_l % 8 == 0
    nhop = -(-max(nfull + (1 if rem else 0),
                  nfull_l + (1 if rem_l else 0)) // NS)

    @functools.partial(
        pl.kernel,
        out_type=jax.ShapeDtypeStruct((NC * N, D), jnp.float32),
        mesh=_mesh(),
        scratch_types=(
            [pltpu.VMEM((CH,), jnp.int32) for _ in range(NB)]
            + [pltpu.VMEM((CH,), jnp.int32) for _ in range(NB)]
            + [pltpu.VMEM((CH,), jnp.float32) for _ in range(NB)]
            + [pltpu.VMEM((CH, D), jnp.float32) for _ in range(NB)]
            + [pltpu.VMEM((max(REM, 8),), jnp.int32)]
            + [pltpu.VMEM_SHARED((HN + 8, D), jnp.float32)]
            + [pltpu.SemaphoreType.DMA for _ in range(4 * NB)]
        ),
    )
    def edge_kernel(y_hbm, row_hbm, col_hbm, ew_hbm, out_hbm, *refs):
        idx_v = refs[0:NB]
        col_v = refs[NB:2 * NB]
        ew_v = refs[2 * NB:3 * NB]
        rows_v = refs[3 * NB:4 * NB]
        col_rem = refs[4 * NB]
        acc_sh = refs[4 * NB + 1]
        i_sem = refs[4 * NB + 2:5 * NB + 2]
        c_sem = refs[5 * NB + 2:6 * NB + 2]
        g_sem = refs[6 * NB + 2:7 * NB + 2]
        s_sem = refs[7 * NB + 2:8 * NB + 2]

        cid = lax.axis_index("c")
        sid = lax.axis_index("s")
        ebase = sid * TPE
        offv = cid * N

        assert NCH % NB == 0

        def issue_idx(kk, b, first):
            if not first:
                @pl.when(kk >= NB)
                def _():
                    pltpu.make_async_copy(
                        rows_v[b], acc_sh.at[col_v[b]], s_sem[b]).wait()
            base = ebase + kk * CH
            pltpu.async_copy(row_hbm.at[pl.ds(base, CH)], idx_v[b], i_sem[b])
            pltpu.async_copy(col_hbm.at[pl.ds(base, CH)], col_v[b], c_sem[b])
            pltpu.async_copy(ew_hbm.at[pl.ds(base, CH)], ew_v[b], c_sem[b])

        def prep(kk, b):
            pltpu.make_async_copy(
                row_hbm.at[pl.ds(ebase + kk * CH, CH)], idx_v[b],
                i_sem[b]).wait()
            for g in range(CH // L):
                sl = pl.ds(g * L, L)
                idx_v[b][sl] = idx_v[b][sl] + offv
            pltpu.async_copy(y_hbm.at[idx_v[b]], rows_v[b], g_sem[b])

        def scale(rbuf, ebuf, nedges):
            def sbody(jg, c2):
                wv = ebuf[pl.ds(jg * L, L)]
                for l in range(L):
                    s = wv[l]
                    j = jg * L + l
                    for k in range(D // L):
                        sl = pl.ds(k * L, L)
                        rbuf[j, sl] = rbuf[j, sl] * s
                return c2
            lax.fori_loop(0, nedges // L, sbody, 0)

        def rebase(cbuf, lo, nedges):
            for g in range(nedges // L):
                sl = pl.ds(g * L, L)
                cv = cbuf[sl] - lo
                cbuf[sl] = jnp.where((cv >= 0) & (cv < HN), cv, HN)

        def pbody(p, carry):
            lo = p * HN
            woff = cid * N + lo

            def process(k, b):
                pltpu.make_async_copy(
                    col_hbm.at[pl.ds(ebase + k * CH, CH)], col_v[b],
                    c_sem[b]).wait()
                pltpu.make_async_copy(
                    ew_hbm.at[pl.ds(ebase + k * CH, CH)], ew_v[b],
                    c_sem[b]).wait()
                pltpu.make_async_copy(y_hbm.at[idx_v[b]], rows_v[b],
                                      g_sem[b]).wait()
                scale(rows_v[b], ew_v[b], CH)
                rebase(col_v[b], lo, CH)
                pltpu.async_copy(rows_v[b], acc_sh.at[col_v[b]], s_sem[b],
                                 add=True)

            def ibody(ih, c):
                h = sid + ih * NS

                @pl.when((h < nfull) & ((p + 1 < NPASS) | (h < nfull_l)))
                def _():
                    r0 = h * 128
                    pltpu.sync_copy(y_hbm.at[pl.ds(woff + r0, 128)],
                                    acc_sh.at[pl.ds(r0, 128)])
                if rem:
                    @pl.when((h == nfull) & (p + 1 < NPASS))
                    def _():
                        r0 = nfull * 128
                        pltpu.sync_copy(y_hbm.at[pl.ds(woff + r0, rem)],
                                        acc_sh.at[pl.ds(r0, rem)])
                if rem_l:
                    @pl.when((h == nfull_l) & (p + 1 == NPASS))
                    def _():
                        r0 = nfull_l * 128
                        pltpu.sync_copy(y_hbm.at[pl.ds(woff + r0, rem_l)],
                                        acc_sh.at[pl.ds(r0, rem_l)])
                return c
            lax.fori_loop(0, nhop, ibody, 0)
            plsc.subcore_barrier()

            issue_idx(0, 0, first=True)
            issue_idx(1, 1, first=True)
            prep(0, 0)

            def mbody(it, c):
                for b in range(NB):
                    k = it * NB + b

                    @pl.when(k + 2 < NCH)
                    def _():
                        issue_idx(k + 2, (b + 2) % NB, first=False)

                    @pl.when(k + 1 < NCH)
                    def _():
                        prep(k + 1, (b + 1) % NB)
                    process(k, b)
                return c
            lax.fori_loop(0, NCH // NB, mbody, 0)

            for b in range(NB):
                pltpu.make_async_copy(rows_v[b], acc_sh.at[col_v[b]],
                                      s_sem[b]).wait()

            if REM:
                base = ebase + NCH * CH
                pltpu.sync_copy(row_hbm.at[pl.ds(base, REM)],
                                idx_v[0].at[pl.ds(0, REM)])
                pltpu.sync_copy(col_hbm.at[pl.ds(base, REM)], col_rem)
                pltpu.sync_copy(ew_hbm.at[pl.ds(base, REM)],
                                ew_v[0].at[pl.ds(0, REM)])
                for g in range(REM // L):
                    sl = pl.ds(g * L, L)
                    idx_v[0][sl] = idx_v[0][sl] + offv
                pltpu.async_copy(y_hbm.at[idx_v[0].at[pl.ds(0, REM)]],
                                 rows_v[0].at[pl.ds(0, REM)], g_sem[0])
                pltpu.make_async_copy(
                    y_hbm.at[idx_v[0].at[pl.ds(0, REM)]],
                    rows_v[0].at[pl.ds(0, REM)], g_sem[0]).wait()
                scale(rows_v[0], ew_v[0], REM)
                rebase(col_rem, lo, REM)
                pltpu.async_copy(rows_v[0].at[pl.ds(0, REM)],
                                 acc_sh.at[col_rem], s_sem[0], add=True)
                pltpu.make_async_copy(rows_v[0].at[pl.ds(0, REM)],
                                      acc_sh.at[col_rem], s_sem[0]).wait()

            plsc.subcore_barrier()

            def obody(ih, c):
                h = sid + ih * NS

                @pl.when((h < nfull) & ((p + 1 < NPASS) | (h < nfull_l)))
                def _():
                    r0 = h * 128
                    pltpu.sync_copy(acc_sh.at[pl.ds(r0, 128)],
                                    out_hbm.at[pl.ds(woff + r0, 128)])
                if rem:
                    @pl.when((h == nfull) & (p + 1 < NPASS))
                    def _():
                        r0 = nfull * 128
                        pltpu.sync_copy(acc_sh.at[pl.ds(r0, rem)],
                                        out_hbm.at[pl.ds(woff + r0, rem)])
                if rem_l:
                    @pl.when((h == nfull_l) & (p + 1 == NPASS))
                    def _():
                        r0 = nfull_l * 128
                        pltpu.sync_copy(acc_sh.at[pl.ds(r0, rem_l)],
                                        out_hbm.at[pl.ds(woff + r0, rem_l)])
                return c
            lax.fori_loop(0, nhop, obody, 0)
            plsc.subcore_barrier()
            return carry

        lax.fori_loop(0, NPASS, pbody, 0)

    return edge_kernel


def _split_q(y_ref, y):
    nq = y_ref.shape[0]
    qd = y.shape[1] // nq
    for q in range(nq):
        y_ref[q] = y[:, q * qd:(q + 1) * qd]


def _cat_q(a_ref):
    return jnp.concatenate([a_ref[q] for q in range(a_ref.shape[0])], axis=1)


def _tc1_body(x_ref, w_ref, degp_ref, y_ref, dis_ref):
    dsum = jnp.sum(degp_ref[...], axis=0) + 1.0
    dis = jnp.where(dsum > 0, lax.rsqrt(dsum), 0.0)[:, None]
    xw = jnp.dot(x_ref[...], w_ref[...], preferred_element_type=jnp.float32)
    _split_q(y_ref, xw * dis)
    dis_ref[...] = dis


def _tc2_body(a_ref, dis_ref, b_ref, w_ref, y_ref):
    d = dis_ref[...]
    hid = jnp.maximum(_cat_q(a_ref) * d + b_ref[...], 0.0)
    y = jnp.dot(hid, w_ref[...], preferred_element_type=jnp.float32) * d
    _split_q(y_ref, y)


def _tc3_body(a_ref, dis_ref, b_ref, w_ref, bl_ref, o_ref):
    d = dis_ref[...]
    hid = jnp.maximum(_cat_q(a_ref) * d + b_ref[...], 0.0)
    o_ref[...] = (
        jnp.dot(hid, w_ref[...], preferred_element_type=jnp.float32) + bl_ref[...]
    )


NQ = 2


def _tc1(x, W, degp, BR):
    N, K = x.shape
    H = W.shape[1]
    NW = degp.shape[0]
    grid = (-(-N // BR),)
    return pl.pallas_call(
        _tc1_body,
        grid=grid,
        in_specs=[
            pl.BlockSpec((BR, K), lambda i: (i, 0)),
            pl.BlockSpec((K, H), lambda i: (0, 0)),
            pl.BlockSpec((NW, BR), lambda i: (0, i)),
        ],
        out_specs=[
            pl.BlockSpec((NQ, BR, H // NQ), lambda i: (0, i, 0)),
            pl.BlockSpec((BR, 1), lambda i: (i, 0)),
        ],
        out_shape=[
            jax.ShapeDtypeStruct((NQ, N, H // NQ), jnp.float32),
            jax.ShapeDtypeStruct((N, 1), jnp.float32),
        ],
    )(x, W, degp)


def _tc2(acc, dis_col, b, W, BR):
    _, N, Hq = acc.shape
    H = W.shape[1]
    grid = (-(-N // BR),)
    return pl.pallas_call(
        _tc2_body,
        grid=grid,
        in_specs=[
            pl.BlockSpec((NQ, BR, Hq), lambda i: (0, i, 0)),
            pl.BlockSpec((BR, 1), lambda i: (i, 0)),
            pl.BlockSpec((1, NQ * Hq), lambda i: (0, 0)),
            pl.BlockSpec((NQ * Hq, H), lambda i: (0, 0)),
        ],
        out_specs=pl.BlockSpec((NQ, BR, H // NQ), lambda i: (0, i, 0)),
        out_shape=jax.ShapeDtypeStruct((NQ, N, H // NQ), jnp.float32),
    )(acc, dis_col, b, W)


def _tc3(acc, dis_col, b, W, bl, BR):
    _, N, Hq = acc.shape
    DO = W.shape[1]
    grid = (-(-N // BR),)
    return pl.pallas_call(
        _tc3_body,
        grid=grid,
        in_specs=[
            pl.BlockSpec((NQ, BR, Hq), lambda i: (0, i, 0)),
            pl.BlockSpec((BR, 1), lambda i: (i, 0)),
            pl.BlockSpec((1, NQ * Hq), lambda i: (0, 0)),
            pl.BlockSpec((NQ * Hq, DO), lambda i: (0, 0)),
            pl.BlockSpec((1, DO), lambda i: (0, 0)),
        ],
        out_specs=pl.BlockSpec((BR, DO), lambda i: (i, 0)),
        out_shape=jax.ShapeDtypeStruct((N, DO), jnp.float32),
    )(acc, dis_col, b, W, bl)


def kernel(x, edge_index, edge_weight, W1, b1, W2, b2, Wl, bl):
    N = x.shape[0]
    E = edge_weight.shape[0]
    BR = 1024

    row = edge_index[0]
    col = edge_index[1]

    deg_fn, NP = _make_deg(E, N)
    degp = deg_fn(col, edge_weight).reshape(NC * NS, NP)

    QD = 256 // NQ
    edge_fn = _make_edge(E, N, QD)

    y1, dis_col = _tc1(x, W1, degp, BR)
    a1 = edge_fn(y1.reshape(NQ * N, QD), row, col, edge_weight)
    y2 = _tc2(a1.reshape(NQ, N, QD), dis_col, b1.reshape(1, -1), W2, BR)
    a2 = edge_fn(y2.reshape(NQ * N, QD), row, col, edge_weight)
    out = _tc3(a2.reshape(NQ, N, QD), dis_col, b2.reshape(1, -1), Wl,
               bl.reshape(1, -1), BR)
    return out


# speedup vs baseline: 18.2317x; 2.0552x over previous
import functools

import jax
import jax.numpy as jnp
from jax import lax
from jax.experimental import pallas as pl
from jax.experimental.pallas import tpu as pltpu
from jax.experimental.pallas import tpu_sc as plsc

NC, NS, L = 2, 16, 16


def _mesh():
    return plsc.VectorSubcoreMesh(
        core_axis_name="c", subcore_axis_name="s", num_cores=NC, num_subcores=NS
    )


def _make_deg(E, N):
    CH = 128
    n_chunks = E // CH
    NW = NC * NS
    cpw = -(-n_chunks // NW)
    NP = (-(-N // 1024)) * 1024

    @functools.partial(
        pl.kernel,
        out_type=jax.ShapeDtypeStruct((NW * NP,), jnp.float32),
        mesh=_mesh(),
        scratch_types=[
            pltpu.VMEM((CH,), jnp.int32),
            pltpu.VMEM((CH,), jnp.float32),
            pltpu.VMEM((NP,), jnp.float32),
        ],
        compiler_params=pltpu.CompilerParams(needs_layout_passes=False),
    )
    def deg_kernel(col_hbm, ew_hbm, out_hbm, col_v, ew_v, deg_v):
        cid = lax.axis_index("c")
        sid = lax.axis_index("s")
        w = sid * NC + cid
        zero16 = jnp.zeros((L,), jnp.float32)

        def zbody(i, c):
            deg_v[pl.ds(i * L, L)] = zero16
            return c
        lax.fori_loop(0, NP // L, zbody, 0)

        def ebody(it, c):
            chunk = w + it * NW

            @pl.when(chunk < n_chunks)
            def _():
                base = chunk * CH
                pltpu.sync_copy(col_hbm.at[pl.ds(base, CH)], col_v)
                pltpu.sync_copy(ew_hbm.at[pl.ds(base, CH)], ew_v)
                for g in range(CH // L):
                    cvec = col_v[pl.ds(g * L, L)]
                    wvec = ew_v[pl.ds(g * L, L)]
                    plsc.addupdate_scatter(deg_v, [cvec], wvec)
            return c
        lax.fori_loop(0, cpw, ebody, 0)

        pltpu.sync_copy(deg_v, out_hbm.at[pl.ds(w * NP, NP)])

    return deg_kernel, NP


def _make_edge(E, N, D):
    CH = 96
    NB = 3
    TPE = E // NS
    NCH = TPE // CH
    REM = TPE - NCH * CH
    NCHP = (NCH // NB) * NB
    assert CH % L == 0 and REM % L == 0
    nfull = N // 128
    rem = N - nfull * 128
    nhop = -(-(nfull + (1 if rem else 0)) // NS)

    @functools.partial(
        pl.kernel,
        out_type=jax.ShapeDtypeStruct((NC * N, D), jnp.float32),
        mesh=_mesh(),
        scratch_types=(
            [pltpu.VMEM((CH,), jnp.int32) for _ in range(NB)]
            + [pltpu.VMEM((CH,), jnp.int32) for _ in range(NB)]
            + [pltpu.VMEM((CH,), jnp.float32) for _ in range(NB)]
            + [pltpu.VMEM((CH, D), jnp.float32) for _ in range(NB)]
            + [pltpu.VMEM((max(REM, 8),), jnp.int32)]
            + [pltpu.VMEM_SHARED((N, D), jnp.float32)]
            + [pltpu.SemaphoreType.DMA for _ in range(4 * NB)]
        ),
    )
    def edge_kernel(y_hbm, row_hbm, col_hbm, ew_hbm, out_hbm, *refs):
        idx_v = refs[0:NB]
        col_v = refs[NB:2 * NB]
        ew_v = refs[2 * NB:3 * NB]
        rows_v = refs[3 * NB:4 * NB]
        col_rem = refs[4 * NB]
        acc_sh = refs[4 * NB + 1]
        i_sem = refs[4 * NB + 2:5 * NB + 2]
        c_sem = refs[5 * NB + 2:6 * NB + 2]
        g_sem = refs[6 * NB + 2:7 * NB + 2]
        s_sem = refs[7 * NB + 2:8 * NB + 2]

        cid = lax.axis_index("c")
        sid = lax.axis_index("s")
        ebase = sid * TPE
        offv = cid * N

        def issue_idx(kk, b, first):
            if not first:
                @pl.when(kk >= NB)
                def _():
                    pltpu.make_async_copy(
                        rows_v[b], acc_sh.at[col_v[b]], s_sem[b]).wait()
            base = ebase + kk * CH
            pltpu.async_copy(row_hbm.at[pl.ds(base, CH)], idx_v[b], i_sem[b])
            pltpu.async_copy(col_hbm.at[pl.ds(base, CH)], col_v[b], c_sem[b])
            pltpu.async_copy(ew_hbm.at[pl.ds(base, CH)], ew_v[b], c_sem[b])

        def prep(kk, b):
            pltpu.make_async_copy(
                row_hbm.at[pl.ds(ebase + kk * CH, CH)], idx_v[b],
                i_sem[b]).wait()
            for g in range(CH // L):
                sl = pl.ds(g * L, L)
                idx_v[b][sl] = idx_v[b][sl] + offv
            pltpu.async_copy(y_hbm.at[idx_v[b]], rows_v[b], g_sem[b])

        def scale(rbuf, ebuf, nedges):
            def sbody(jg, c2):
                wv = ebuf[pl.ds(jg * L, L)]
                for l in range(L):
                    s = wv[l]
                    j = jg * L + l
                    for k in range(D // L):
                        sl = pl.ds(k * L, L)
                        rbuf[j, sl] = rbuf[j, sl] * s
                return c2
            lax.fori_loop(0, nedges // L, sbody, 0)

        def process(k, b):
            pltpu.make_async_copy(
                col_hbm.at[pl.ds(ebase + k * CH, CH)], col_v[b],
                c_sem[b]).wait()
            pltpu.make_async_copy(
                ew_hbm.at[pl.ds(ebase + k * CH, CH)], ew_v[b],
                c_sem[b]).wait()
            pltpu.make_async_copy(y_hbm.at[idx_v[b]], rows_v[b],
                                  g_sem[b]).wait()
            scale(rows_v[b], ew_v[b], CH)
            pltpu.async_copy(rows_v[b], acc_sh.at[col_v[b]], s_sem[b],
                             add=True)

        def ibody(ih, c):
            h = sid + ih * NS

            @pl.when(h < nfull)
            def _():
                r0 = h * 128
                pltpu.sync_copy(y_hbm.at[pl.ds(cid * N + r0, 128)],
                                acc_sh.at[pl.ds(r0, 128)])
            if rem:
                @pl.when(h == nfull)
                def _():
                    r0 = nfull * 128
                    pltpu.sync_copy(y_hbm.at[pl.ds(cid * N + r0, rem)],
                                    acc_sh.at[pl.ds(r0, rem)])
            return c
        lax.fori_loop(0, nhop, ibody, 0)
        plsc.subcore_barrier()

        issue_idx(0, 0, first=True)
        issue_idx(1, 1, first=True)
        prep(0, 0)

        def mbody(it, c):
            for b in range(NB):
                k = it * NB + b

                @pl.when(k + 1 < NCHP)
                def _():
                    prep(k + 1, (b + 1) % NB)
                process(k, b)

                @pl.when(k + 2 < NCHP)
                def _():
                    issue_idx(k + 2, (b + 2) % NB, first=False)
            return c
        lax.fori_loop(0, NCHP // NB, mbody, 0)

        for b in range(NB):
            pltpu.make_async_copy(rows_v[b], acc_sh.at[col_v[b]],
                                  s_sem[b]).wait()

        for kx in range(NCHP, NCH):
            base = ebase + kx * CH
            pltpu.sync_copy(row_hbm.at[pl.ds(base, CH)], idx_v[0])
            pltpu.sync_copy(col_hbm.at[pl.ds(base, CH)], col_v[0])
            pltpu.sync_copy(ew_hbm.at[pl.ds(base, CH)], ew_v[0])
            for g in range(CH // L):
                sl = pl.ds(g * L, L)
                idx_v[0][sl] = idx_v[0][sl] + offv
            pltpu.async_copy(y_hbm.at[idx_v[0]], rows_v[0], g_sem[0])
            pltpu.make_async_copy(y_hbm.at[idx_v[0]], rows_v[0],
                                  g_sem[0]).wait()
            scale(rows_v[0], ew_v[0], CH)
            pltpu.async_copy(rows_v[0], acc_sh.at[col_v[0]], s_sem[0],
                             add=True)
            pltpu.make_async_copy(rows_v[0], acc_sh.at[col_v[0]],
                                  s_sem[0]).wait()

        if REM:
            base = ebase + NCH * CH
            pltpu.sync_copy(row_hbm.at[pl.ds(base, REM)],
                            idx_v[0].at[pl.ds(0, REM)])
            pltpu.sync_copy(col_hbm.at[pl.ds(base, REM)], col_rem)
            pltpu.sync_copy(ew_hbm.at[pl.ds(base, REM)],
                            ew_v[0].at[pl.ds(0, REM)])
            for g in range(REM // L):
                sl = pl.ds(g * L, L)
                idx_v[0][sl] = idx_v[0][sl] + offv
            pltpu.async_copy(y_hbm.at[idx_v[0].at[pl.ds(0, REM)]],
                             rows_v[0].at[pl.ds(0, REM)], g_sem[0])
            pltpu.make_async_copy(y_hbm.at[idx_v[0].at[pl.ds(0, REM)]],
                                  rows_v[0].at[pl.ds(0, REM)], g_sem[0]).wait()
            scale(rows_v[0], ew_v[0], REM)
            pltpu.async_copy(rows_v[0].at[pl.ds(0, REM)],
                             acc_sh.at[col_rem], s_sem[0], add=True)
            pltpu.make_async_copy(rows_v[0].at[pl.ds(0, REM)],
                                  acc_sh.at[col_rem], s_sem[0]).wait()

        plsc.subcore_barrier()

        def obody(ih, c):
            h = sid + ih * NS

            @pl.when(h < nfull)
            def _():
                r0 = h * 128
                pltpu.sync_copy(acc_sh.at[pl.ds(r0, 128)],
                                out_hbm.at[pl.ds(cid * N + r0, 128)])
            if rem:
                @pl.when(h == nfull)
                def _():
                    r0 = nfull * 128
                    pltpu.sync_copy(acc_sh.at[pl.ds(r0, rem)],
                                    out_hbm.at[pl.ds(cid * N + r0, rem)])
            return c
        lax.fori_loop(0, nhop, obody, 0)

    return edge_kernel


def _split_q(y_ref, y):
    nq = y_ref.shape[0]
    qd = y.shape[1] // nq
    for q in range(nq):
        y_ref[q] = y[:, q * qd:(q + 1) * qd]


def _cat_q(a_ref):
    return jnp.concatenate([a_ref[q] for q in range(a_ref.shape[0])], axis=1)


def _tc1_body(x_ref, w_ref, degp_ref, y_ref, dis_ref):
    dsum = jnp.sum(degp_ref[...], axis=0) + 1.0
    dis = jnp.where(dsum > 0, lax.rsqrt(dsum), 0.0)[:, None]
    xw = jnp.dot(x_ref[...], w_ref[...], preferred_element_type=jnp.float32)
    _split_q(y_ref, xw * dis)
    dis_ref[...] = dis


def _tc2_body(a_ref, dis_ref, b_ref, w_ref, y_ref):
    d = dis_ref[...]
    hid = jnp.maximum(_cat_q(a_ref) * d + b_ref[...], 0.0)
    y = jnp.dot(hid, w_ref[...], preferred_element_type=jnp.float32) * d
    _split_q(y_ref, y)


def _tc3_body(a_ref, dis_ref, b_ref, w_ref, bl_ref, o_ref):
    d = dis_ref[...]
    hid = jnp.maximum(_cat_q(a_ref) * d + b_ref[...], 0.0)
    o_ref[...] = (
        jnp.dot(hid, w_ref[...], preferred_element_type=jnp.float32) + bl_ref[...]
    )


NQ = 2


def _tc1(x, W, degp, BR):
    N, K = x.shape
    H = W.shape[1]
    NW = degp.shape[0]
    grid = (-(-N // BR),)
    return pl.pallas_call(
        _tc1_body,
        grid=grid,
        in_specs=[
            pl.BlockSpec((BR, K), lambda i: (i, 0)),
            pl.BlockSpec((K, H), lambda i: (0, 0)),
            pl.BlockSpec((NW, BR), lambda i: (0, i)),
        ],
        out_specs=[
            pl.BlockSpec((NQ, BR, H // NQ), lambda i: (0, i, 0)),
            pl.BlockSpec((BR, 1), lambda i: (i, 0)),
        ],
        out_shape=[
            jax.ShapeDtypeStruct((NQ, N, H // NQ), jnp.float32),
            jax.ShapeDtypeStruct((N, 1), jnp.float32),
        ],
    )(x, W, degp)


def _tc2(acc, dis_col, b, W, BR):
    _, N, Hq = acc.shape
    H = W.shape[1]
    grid = (-(-N // BR),)
    return pl.pallas_call(
        _tc2_body,
        grid=grid,
        in_specs=[
            pl.BlockSpec((NQ, BR, Hq), lambda i: (0, i, 0)),
            pl.BlockSpec((BR, 1), lambda i: (i, 0)),
            pl.BlockSpec((1, NQ * Hq), lambda i: (0, 0)),
            pl.BlockSpec((NQ * Hq, H), lambda i: (0, 0)),
        ],
        out_specs=pl.BlockSpec((NQ, BR, H // NQ), lambda i: (0, i, 0)),
        out_shape=jax.ShapeDtypeStruct((NQ, N, H // NQ), jnp.float32),
    )(acc, dis_col, b, W)


def _tc3(acc, dis_col, b, W, bl, BR):
    _, N, Hq = acc.shape
    DO = W.shape[1]
    grid = (-(-N // BR),)
    return pl.pallas_call(
        _tc3_body,
        grid=grid,
        in_specs=[
            pl.BlockSpec((NQ, BR, Hq), lambda i: (0, i, 0)),
            pl.BlockSpec((BR, 1), lambda i: (i, 0)),
            pl.BlockSpec((1, NQ * Hq), lambda i: (0, 0)),
            pl.BlockSpec((NQ * Hq, DO), lambda i: (0, 0)),
            pl.BlockSpec((1, DO), lambda i: (0, 0)),
        ],
        out_specs=pl.BlockSpec((BR, DO), lambda i: (i, 0)),
        out_shape=jax.ShapeDtypeStruct((N, DO), jnp.float32),
    )(acc, dis_col, b, W, bl)


def kernel(x, edge_index, edge_weight, W1, b1, W2, b2, Wl, bl):
    N = x.shape[0]
    E = edge_weight.shape[0]
    BR = 1024

    row = edge_index[0]
    col = edge_index[1]

    deg_fn, NP = _make_deg(E, N)
    degp = deg_fn(col, edge_weight).reshape(NC * NS, NP)

    QD = 256 // NQ
    edge_fn = _make_edge(E, N, QD)

    y1, dis_col = _tc1(x, W1, degp, BR)
    a1 = edge_fn(y1.reshape(NQ * N, QD), row, col, edge_weight)
    y2 = _tc2(a1.reshape(NQ, N, QD), dis_col, b1.reshape(1, -1), W2, BR)
    a2 = edge_fn(y2.reshape(NQ * N, QD), row, col, edge_weight)
    out = _tc3(a2.reshape(NQ, N, QD), dis_col, b2.reshape(1, -1), Wl,
               bl.reshape(1, -1), BR)
    return out


# speedup vs baseline: 19.1982x; 1.0530x over previous
import functools

import jax
import jax.numpy as jnp
from jax import lax
from jax.experimental import pallas as pl
from jax.experimental.pallas import tpu as pltpu
from jax.experimental.pallas import tpu_sc as plsc

NC, NS, L = 2, 16, 16


def _mesh():
    return plsc.VectorSubcoreMesh(
        core_axis_name="c", subcore_axis_name="s", num_cores=NC, num_subcores=NS
    )


def _make_deg(E, N):
    NW = NC * NS
    TPW = E // NW
    assert TPW % L == 0
    NP = (-(-N // 1024)) * 1024

    @functools.partial(
        pl.kernel,
        out_type=jax.ShapeDtypeStruct((NW * NP,), jnp.float32),
        mesh=_mesh(),
        scratch_types=[
            pltpu.VMEM((TPW,), jnp.int32),
            pltpu.VMEM((TPW,), jnp.float32),
            pltpu.VMEM((NP,), jnp.float32),
        ],
        compiler_params=pltpu.CompilerParams(needs_layout_passes=False),
    )
    def deg_kernel(col_hbm, ew_hbm, out_hbm, col_v, ew_v, deg_v):
        cid = lax.axis_index("c")
        sid = lax.axis_index("s")
        w = sid * NC + cid
        zero16 = jnp.zeros((L,), jnp.float32)
        pltpu.sync_copy(col_hbm.at[pl.ds(w * TPW, TPW)], col_v)
        pltpu.sync_copy(ew_hbm.at[pl.ds(w * TPW, TPW)], ew_v)

        def zbody(i, c):
            deg_v[pl.ds(i * L, L)] = zero16
            return c
        lax.fori_loop(0, NP // L, zbody, 0)

        def ebody(g, c):
            sl = pl.ds(g * L, L)
            plsc.addupdate_scatter(deg_v, [col_v[sl]], ew_v[sl])
            return c
        lax.fori_loop(0, TPW // L, ebody, 0)

        pltpu.sync_copy(deg_v, out_hbm.at[pl.ds(w * NP, NP)])

    return deg_kernel, NP


def _make_edge(E, N, D):
    CH = 80
    NB = 4
    TPE = E // NS
    NCH = TPE // CH
    REM = TPE - NCH * CH
    NCHP = (NCH // NB) * NB
    assert CH % L == 0 and REM % L == 0
    nfull = N // 128
    rem = N - nfull * 128
    nhop = -(-(nfull + (1 if rem else 0)) // NS)

    @functools.partial(
        pl.kernel,
        out_type=jax.ShapeDtypeStruct((NC * N, D), jnp.float32),
        mesh=_mesh(),
        scratch_types=(
            [pltpu.VMEM((CH,), jnp.int32) for _ in range(NB)]
            + [pltpu.VMEM((CH,), jnp.int32) for _ in range(NB)]
            + [pltpu.VMEM((CH,), jnp.float32) for _ in range(NB)]
            + [pltpu.VMEM((CH, D), jnp.float32) for _ in range(NB)]
            + [pltpu.VMEM((max(REM, 8),), jnp.int32)]
            + [pltpu.VMEM_SHARED((N, D), jnp.float32)]
            + [pltpu.SemaphoreType.DMA for _ in range(4 * NB)]
        ),
    )
    def edge_kernel(y_hbm, row_hbm, col_hbm, ew_hbm, out_hbm, *refs):
        idx_v = refs[0:NB]
        col_v = refs[NB:2 * NB]
        ew_v = refs[2 * NB:3 * NB]
        rows_v = refs[3 * NB:4 * NB]
        col_rem = refs[4 * NB]
        acc_sh = refs[4 * NB + 1]
        i_sem = refs[4 * NB + 2:5 * NB + 2]
        c_sem = refs[5 * NB + 2:6 * NB + 2]
        g_sem = refs[6 * NB + 2:7 * NB + 2]
        s_sem = refs[7 * NB + 2:8 * NB + 2]

        cid = lax.axis_index("c")
        sid = lax.axis_index("s")
        ebase = sid * TPE
        offv = cid * N

        def issue_idx(kk, b, first):
            if not first:
                @pl.when(kk >= NB)
                def _():
                    pltpu.make_async_copy(
                        rows_v[b], acc_sh.at[col_v[b]], s_sem[b]).wait()
            base = ebase + kk * CH
            pltpu.async_copy(row_hbm.at[pl.ds(base, CH)], idx_v[b], i_sem[b])
            pltpu.async_copy(col_hbm.at[pl.ds(base, CH)], col_v[b], c_sem[b])
            pltpu.async_copy(ew_hbm.at[pl.ds(base, CH)], ew_v[b], c_sem[b])

        def prep(kk, b):
            pltpu.make_async_copy(
                row_hbm.at[pl.ds(ebase + kk * CH, CH)], idx_v[b],
                i_sem[b]).wait()
            for g in range(CH // L):
                sl = pl.ds(g * L, L)
                idx_v[b][sl] = idx_v[b][sl] + offv
            pltpu.async_copy(y_hbm.at[idx_v[b]], rows_v[b], g_sem[b])

        def scale(rbuf, ebuf, nedges):
            def sbody(jg, c2):
                wv = ebuf[pl.ds(jg * L, L)]
                for l in range(L):
                    s = wv[l]
                    j = jg * L + l
                    for k in range(D // L):
                        sl = pl.ds(k * L, L)
                        rbuf[j, sl] = rbuf[j, sl] * s
                return c2
            lax.fori_loop(0, nedges // L, sbody, 0)

        def process(k, b):
            pltpu.make_async_copy(
                col_hbm.at[pl.ds(ebase + k * CH, CH)], col_v[b],
                c_sem[b]).wait()
            pltpu.make_async_copy(
                ew_hbm.at[pl.ds(ebase + k * CH, CH)], ew_v[b],
                c_sem[b]).wait()
            pltpu.make_async_copy(y_hbm.at[idx_v[b]], rows_v[b],
                                  g_sem[b]).wait()
            scale(rows_v[b], ew_v[b], CH)
            pltpu.async_copy(rows_v[b], acc_sh.at[col_v[b]], s_sem[b],
                             add=True)

        def ibody(ih, c):
            h = sid + ih * NS

            @pl.when(h < nfull)
            def _():
                r0 = h * 128
                pltpu.sync_copy(y_hbm.at[pl.ds(cid * N + r0, 128)],
                                acc_sh.at[pl.ds(r0, 128)])
            if rem:
                @pl.when(h == nfull)
                def _():
                    r0 = nfull * 128
                    pltpu.sync_copy(y_hbm.at[pl.ds(cid * N + r0, rem)],
                                    acc_sh.at[pl.ds(r0, rem)])
            return c
        lax.fori_loop(0, nhop, ibody, 0)
        plsc.subcore_barrier()

        issue_idx(0, 0, first=True)
        issue_idx(1, 1, first=True)
        prep(0, 0)

        def mbody(it, c):
            for b in range(NB):
                k = it * NB + b

                @pl.when(k + 1 < NCHP)
                def _():
                    prep(k + 1, (b + 1) % NB)
                process(k, b)

                @pl.when(k + 2 < NCHP)
                def _():
                    issue_idx(k + 2, (b + 2) % NB, first=False)
            return c
        lax.fori_loop(0, NCHP // NB, mbody, 0)

        for b in range(NB):
            pltpu.make_async_copy(rows_v[b], acc_sh.at[col_v[b]],
                                  s_sem[b]).wait()

        for kx in range(NCHP, NCH):
            base = ebase + kx * CH
            pltpu.sync_copy(row_hbm.at[pl.ds(base, CH)], idx_v[0])
            pltpu.sync_copy(col_hbm.at[pl.ds(base, CH)], col_v[0])
            pltpu.sync_copy(ew_hbm.at[pl.ds(base, CH)], ew_v[0])
            for g in range(CH // L):
                sl = pl.ds(g * L, L)
                idx_v[0][sl] = idx_v[0][sl] + offv
            pltpu.async_copy(y_hbm.at[idx_v[0]], rows_v[0], g_sem[0])
            pltpu.make_async_copy(y_hbm.at[idx_v[0]], rows_v[0],
                                  g_sem[0]).wait()
            scale(rows_v[0], ew_v[0], CH)
            pltpu.async_copy(rows_v[0], acc_sh.at[col_v[0]], s_sem[0],
                             add=True)
            pltpu.make_async_copy(rows_v[0], acc_sh.at[col_v[0]],
                                  s_sem[0]).wait()

        if REM:
            base = ebase + NCH * CH
            pltpu.sync_copy(row_hbm.at[pl.ds(base, REM)],
                            idx_v[0].at[pl.ds(0, REM)])
            pltpu.sync_copy(col_hbm.at[pl.ds(base, REM)], col_rem)
            pltpu.sync_copy(ew_hbm.at[pl.ds(base, REM)],
                            ew_v[0].at[pl.ds(0, REM)])
            for g in range(REM // L):
                sl = pl.ds(g * L, L)
                idx_v[0][sl] = idx_v[0][sl] + offv
            pltpu.async_copy(y_hbm.at[idx_v[0].at[pl.ds(0, REM)]],
                             rows_v[0].at[pl.ds(0, REM)], g_sem[0])
            pltpu.make_async_copy(y_hbm.at[idx_v[0].at[pl.ds(0, REM)]],
                                  rows_v[0].at[pl.ds(0, REM)], g_sem[0]).wait()
            scale(rows_v[0], ew_v[0], REM)
            pltpu.async_copy(rows_v[0].at[pl.ds(0, REM)],
                             acc_sh.at[col_rem], s_sem[0], add=True)
            pltpu.make_async_copy(rows_v[0].at[pl.ds(0, REM)],
                                  acc_sh.at[col_rem], s_sem[0]).wait()

        plsc.subcore_barrier()

        def obody(ih, c):
            h = sid + ih * NS

            @pl.when(h < nfull)
            def _():
                r0 = h * 128
                pltpu.sync_copy(acc_sh.at[pl.ds(r0, 128)],
                                out_hbm.at[pl.ds(cid * N + r0, 128)])
            if rem:
                @pl.when(h == nfull)
                def _():
                    r0 = nfull * 128
                    pltpu.sync_copy(acc_sh.at[pl.ds(r0, rem)],
                                    out_hbm.at[pl.ds(cid * N + r0, rem)])
            return c
        lax.fori_loop(0, nhop, obody, 0)

    return edge_kernel


def _split_q(y_ref, y):
    nq = y_ref.shape[0]
    qd = y.shape[1] // nq
    for q in range(nq):
        y_ref[q] = y[:, q * qd:(q + 1) * qd]


def _cat_q(a_ref):
    return jnp.concatenate([a_ref[q] for q in range(a_ref.shape[0])], axis=1)


def _tc1_body(x_ref, w_ref, degp_ref, y_ref, dis_ref):
    dsum = jnp.sum(degp_ref[...], axis=0) + 1.0
    dis = jnp.where(dsum > 0, lax.rsqrt(dsum), 0.0)[:, None]
    xw = jnp.dot(x_ref[...], w_ref[...], preferred_element_type=jnp.float32)
    _split_q(y_ref, xw * dis)
    dis_ref[...] = dis


def _tc2_body(a_ref, dis_ref, b_ref, w_ref, y_ref):
    d = dis_ref[...]
    hid = jnp.maximum(_cat_q(a_ref) * d + b_ref[...], 0.0)
    y = jnp.dot(hid, w_ref[...], preferred_element_type=jnp.float32) * d
    _split_q(y_ref, y)


def _tc3_body(a_ref, dis_ref, b_ref, w_ref, bl_ref, o_ref):
    d = dis_ref[...]
    hid = jnp.maximum(_cat_q(a_ref) * d + b_ref[...], 0.0)
    o_ref[...] = (
        jnp.dot(hid, w_ref[...], preferred_element_type=jnp.float32) + bl_ref[...]
    )


NQ = 2


def _tc1(x, W, degp, BR):
    N, K = x.shape
    H = W.shape[1]
    NW = degp.shape[0]
    grid = (-(-N // BR),)
    return pl.pallas_call(
        _tc1_body,
        grid=grid,
        in_specs=[
            pl.BlockSpec((BR, K), lambda i: (i, 0)),
            pl.BlockSpec((K, H), lambda i: (0, 0)),
            pl.BlockSpec((NW, BR), lambda i: (0, i)),
        ],
        out_specs=[
            pl.BlockSpec((NQ, BR, H // NQ), lambda i: (0, i, 0)),
            pl.BlockSpec((BR, 1), lambda i: (i, 0)),
        ],
        out_shape=[
            jax.ShapeDtypeStruct((NQ, N, H // NQ), jnp.float32),
            jax.ShapeDtypeStruct((N, 1), jnp.float32),
        ],
    )(x, W, degp)


def _tc2(acc, dis_col, b, W, BR):
    _, N, Hq = acc.shape
    H = W.shape[1]
    grid = (-(-N // BR),)
    return pl.pallas_call(
        _tc2_body,
        grid=grid,
        in_specs=[
            pl.BlockSpec((NQ, BR, Hq), lambda i: (0, i, 0)),
            pl.BlockSpec((BR, 1), lambda i: (i, 0)),
            pl.BlockSpec((1, NQ * Hq), lambda i: (0, 0)),
            pl.BlockSpec((NQ * Hq, H), lambda i: (0, 0)),
        ],
        out_specs=pl.BlockSpec((NQ, BR, H // NQ), lambda i: (0, i, 0)),
        out_shape=jax.ShapeDtypeStruct((NQ, N, H // NQ), jnp.float32),
    )(acc, dis_col, b, W)


def _tc3(acc, dis_col, b, W, bl, BR):
    _, N, Hq = acc.shape
    DO = W.shape[1]
    grid = (-(-N // BR),)
    return pl.pallas_call(
        _tc3_body,
        grid=grid,
        in_specs=[
            pl.BlockSpec((NQ, BR, Hq), lambda i: (0, i, 0)),
            pl.BlockSpec((BR, 1), lambda i: (i, 0)),
            pl.BlockSpec((1, NQ * Hq), lambda i: (0, 0)),
            pl.BlockSpec((NQ * Hq, DO), lambda i: (0, 0)),
            pl.BlockSpec((1, DO), lambda i: (0, 0)),
        ],
        out_specs=pl.BlockSpec((BR, DO), lambda i: (i, 0)),
        out_shape=jax.ShapeDtypeStruct((N, DO), jnp.float32),
    )(acc, dis_col, b, W, bl)


def kernel(x, edge_index, edge_weight, W1, b1, W2, b2, Wl, bl):
    N = x.shape[0]
    E = edge_weight.shape[0]
    BR = 1024

    row = edge_index[0]
    col = edge_index[1]

    deg_fn, NP = _make_deg(E, N)
    degp = deg_fn(col, edge_weight).reshape(NC * NS, NP)

    QD = 256 // NQ
    edge_fn = _make_edge(E, N, QD)

    y1, dis_col = _tc1(x, W1, degp, BR)
    a1 = edge_fn(y1.reshape(NQ * N, QD), row, col, edge_weight)
    y2 = _tc2(a1.reshape(NQ, N, QD), dis_col, b1.reshape(1, -1), W2, BR)
    a2 = edge_fn(y2.reshape(NQ * N, QD), row, col, edge_weight)
    out = _tc3(a2.reshape(NQ, N, QD), dis_col, b2.reshape(1, -1), Wl,
               bl.reshape(1, -1), BR)
    return out


# speedup vs baseline: 20.4384x; 1.0646x over previous
import functools

import jax
import jax.numpy as jnp
from jax import lax
from jax.experimental import pallas as pl
from jax.experimental.pallas import tpu as pltpu
from jax.experimental.pallas import tpu_sc as plsc

NC, NS, L = 2, 16, 16


def _mesh():
    return plsc.VectorSubcoreMesh(
        core_axis_name="c", subcore_axis_name="s", num_cores=NC, num_subcores=NS
    )


def _make_deg(E, N):
    NW = NC * NS
    TPW = E // NW
    assert TPW % L == 0
    NP = (-(-N // 1024)) * 1024

    @functools.partial(
        pl.kernel,
        out_type=jax.ShapeDtypeStruct((NW * NP,), jnp.float32),
        mesh=_mesh(),
        scratch_types=[
            pltpu.VMEM((TPW,), jnp.int32),
            pltpu.VMEM((TPW,), jnp.float32),
            pltpu.VMEM((NP,), jnp.float32),
        ],
        compiler_params=pltpu.CompilerParams(needs_layout_passes=False),
    )
    def deg_kernel(col_hbm, ew_hbm, out_hbm, col_v, ew_v, deg_v):
        cid = lax.axis_index("c")
        sid = lax.axis_index("s")
        w = sid * NC + cid
        zero16 = jnp.zeros((L,), jnp.float32)
        pltpu.sync_copy(col_hbm.at[pl.ds(w * TPW, TPW)], col_v)
        pltpu.sync_copy(ew_hbm.at[pl.ds(w * TPW, TPW)], ew_v)

        def zbody(i, c):
            deg_v[pl.ds(i * L, L)] = zero16
            return c
        lax.fori_loop(0, NP // L, zbody, 0)

        def ebody(g, c):
            sl = pl.ds(g * L, L)
            plsc.addupdate_scatter(deg_v, [col_v[sl]], ew_v[sl])
            return c
        lax.fori_loop(0, TPW // L, ebody, 0)

        pltpu.sync_copy(deg_v, out_hbm.at[pl.ds(w * NP, NP)])

    return deg_kernel, NP


def _make_edge(E, N, D):
    CH = 96
    NB = 3
    TPE = E // NS
    NCH = TPE // CH
    REM = TPE - NCH * CH
    NCHP = (NCH // NB) * NB
    assert CH % L == 0 and REM % L == 0
    nfull = N // 128
    rem = N - nfull * 128
    nhop = -(-(nfull + (1 if rem else 0)) // NS)

    @functools.partial(
        pl.kernel,
        out_type=jax.ShapeDtypeStruct((NC * N, D), jnp.float32),
        mesh=_mesh(),
        scratch_types=(
            [pltpu.VMEM((CH,), jnp.int32) for _ in range(NB)]
            + [pltpu.VMEM((CH,), jnp.int32) for _ in range(NB)]
            + [pltpu.VMEM((CH,), jnp.float32) for _ in range(NB)]
            + [pltpu.VMEM((CH, D), jnp.float32) for _ in range(NB)]
            + [pltpu.VMEM((max(REM, 8),), jnp.int32)]
            + [pltpu.VMEM_SHARED((N, D), jnp.float32)]
            + [pltpu.SemaphoreType.DMA for _ in range(4 * NB)]
        ),
    )
    def edge_kernel(y_hbm, row_hbm, col_hbm, ew_hbm, out_hbm, *refs):
        idx_v = refs[0:NB]
        col_v = refs[NB:2 * NB]
        ew_v = refs[2 * NB:3 * NB]
        rows_v = refs[3 * NB:4 * NB]
        col_rem = refs[4 * NB]
        acc_sh = refs[4 * NB + 1]
        i_sem = refs[4 * NB + 2:5 * NB + 2]
        c_sem = refs[5 * NB + 2:6 * NB + 2]
        g_sem = refs[6 * NB + 2:7 * NB + 2]
        s_sem = refs[7 * NB + 2:8 * NB + 2]

        cid = lax.axis_index("c")
        sid = lax.axis_index("s")
        ebase = sid * TPE
        offv = cid * N

        def issue_idx(kk, b, first):
            if not first:
                @pl.when(kk >= NB)
                def _():
                    pltpu.make_async_copy(
                        rows_v[b], acc_sh.at[col_v[b]], s_sem[b]).wait()
            base = ebase + kk * CH
            pltpu.async_copy(row_hbm.at[pl.ds(base, CH)], idx_v[b], i_sem[b])
            pltpu.async_copy(col_hbm.at[pl.ds(base, CH)], col_v[b], c_sem[b])
            pltpu.async_copy(ew_hbm.at[pl.ds(base, CH)], ew_v[b], c_sem[b])

        def prep(kk, b):
            pltpu.make_async_copy(
                row_hbm.at[pl.ds(ebase + kk * CH, CH)], idx_v[b],
                i_sem[b]).wait()
            for g in range(CH // L):
                sl = pl.ds(g * L, L)
                idx_v[b][sl] = idx_v[b][sl] + offv
            pltpu.async_copy(y_hbm.at[idx_v[b]], rows_v[b], g_sem[b])

        def scale(rbuf, ebuf, nedges):
            def sbody(jg, c2):
                wv = ebuf[pl.ds(jg * L, L)]
                for l in range(L):
                    s = wv[l]
                    j = jg * L + l
                    for k in range(D // L):
                        sl = pl.ds(k * L, L)
                        rbuf[j, sl] = rbuf[j, sl] * s
                return c2
            lax.fori_loop(0, nedges // L, sbody, 0)

        def process(k, b):
            pltpu.make_async_copy(
                col_hbm.at[pl.ds(ebase + k * CH, CH)], col_v[b],
                c_sem[b]).wait()
            pltpu.make_async_copy(
                ew_hbm.at[pl.ds(ebase + k * CH, CH)], ew_v[b],
                c_sem[b]).wait()
            pltpu.make_async_copy(y_hbm.at[idx_v[b]], rows_v[b],
                                  g_sem[b]).wait()
            scale(rows_v[b], ew_v[b], CH)
            pltpu.async_copy(rows_v[b], acc_sh.at[col_v[b]], s_sem[b],
                             add=True)

        def ibody(ih, c):
            h = sid + ih * NS

            @pl.when(h < nfull)
            def _():
                r0 = h * 128
                pltpu.sync_copy(y_hbm.at[pl.ds(cid * N + r0, 128)],
                                acc_sh.at[pl.ds(r0, 128)])
            if rem:
                @pl.when(h == nfull)
                def _():
                    r0 = nfull * 128
                    pltpu.sync_copy(y_hbm.at[pl.ds(cid * N + r0, rem)],
                                    acc_sh.at[pl.ds(r0, rem)])
            return c
        lax.fori_loop(0, nhop, ibody, 0)
        plsc.subcore_barrier()

        issue_idx(0, 0, first=True)
        issue_idx(1, 1, first=True)
        prep(0, 0)

        def mbody(it, c):
            for b in range(NB):
                k = it * NB + b

                @pl.when(k + 1 < NCHP)
                def _():
                    prep(k + 1, (b + 1) % NB)
                process(k, b)

                @pl.when(k + 2 < NCHP)
                def _():
                    issue_idx(k + 2, (b + 2) % NB, first=False)
            return c
        lax.fori_loop(0, NCHP // NB, mbody, 0)

        for b in range(NB):
            pltpu.make_async_copy(rows_v[b], acc_sh.at[col_v[b]],
                                  s_sem[b]).wait()

        for kx in range(NCHP, NCH):
            base = ebase + kx * CH
            pltpu.sync_copy(row_hbm.at[pl.ds(base, CH)], idx_v[0])
            pltpu.sync_copy(col_hbm.at[pl.ds(base, CH)], col_v[0])
            pltpu.sync_copy(ew_hbm.at[pl.ds(base, CH)], ew_v[0])
            for g in range(CH // L):
                sl = pl.ds(g * L, L)
                idx_v[0][sl] = idx_v[0][sl] + offv
            pltpu.async_copy(y_hbm.at[idx_v[0]], rows_v[0], g_sem[0])
            pltpu.make_async_copy(y_hbm.at[idx_v[0]], rows_v[0],
                                  g_sem[0]).wait()
            scale(rows_v[0], ew_v[0], CH)
            pltpu.async_copy(rows_v[0], acc_sh.at[col_v[0]], s_sem[0],
                             add=True)
            pltpu.make_async_copy(rows_v[0], acc_sh.at[col_v[0]],
                                  s_sem[0]).wait()

        if REM:
            base = ebase + NCH * CH
            pltpu.sync_copy(row_hbm.at[pl.ds(base, REM)],
                            idx_v[0].at[pl.ds(0, REM)])
            pltpu.sync_copy(col_hbm.at[pl.ds(base, REM)], col_rem)
            pltpu.sync_copy(ew_hbm.at[pl.ds(base, REM)],
                            ew_v[0].at[pl.ds(0, REM)])
            for g in range(REM // L):
                sl = pl.ds(g * L, L)
                idx_v[0][sl] = idx_v[0][sl] + offv
            pltpu.async_copy(y_hbm.at[idx_v[0].at[pl.ds(0, REM)]],
                             rows_v[0].at[pl.ds(0, REM)], g_sem[0])
            pltpu.make_async_copy(y_hbm.at[idx_v[0].at[pl.ds(0, REM)]],
                                  rows_v[0].at[pl.ds(0, REM)], g_sem[0]).wait()
            scale(rows_v[0], ew_v[0], REM)
            pltpu.async_copy(rows_v[0].at[pl.ds(0, REM)],
                             acc_sh.at[col_rem], s_sem[0], add=True)
            pltpu.make_async_copy(rows_v[0].at[pl.ds(0, REM)],
                                  acc_sh.at[col_rem], s_sem[0]).wait()

        plsc.subcore_barrier()

        def obody(ih, c):
            h = sid + ih * NS

            @pl.when(h < nfull)
            def _():
                r0 = h * 128
                pltpu.sync_copy(acc_sh.at[pl.ds(r0, 128)],
                                out_hbm.at[pl.ds(cid * N + r0, 128)])
            if rem:
                @pl.when(h == nfull)
                def _():
                    r0 = nfull * 128
                    pltpu.sync_copy(acc_sh.at[pl.ds(r0, rem)],
                                    out_hbm.at[pl.ds(cid * N + r0, rem)])
            return c
        lax.fori_loop(0, nhop, obody, 0)

    return edge_kernel


def _split_q(y_ref, y):
    nq = y_ref.shape[0]
    qd = y.shape[1] // nq
    for q in range(nq):
        y_ref[q] = y[:, q * qd:(q + 1) * qd]


def _cat_q(a_ref):
    return jnp.concatenate([a_ref[q] for q in range(a_ref.shape[0])], axis=1)


def _tc1_body(x_ref, w_ref, degp_ref, y_ref, dis_ref):
    dsum = jnp.sum(degp_ref[...], axis=0) + 1.0
    dis = jnp.where(dsum > 0, lax.rsqrt(dsum), 0.0)[:, None]
    xw = jnp.dot(x_ref[...], w_ref[...], preferred_element_type=jnp.float32)
    _split_q(y_ref, xw * dis)
    dis_ref[...] = dis


def _tc2_body(a_ref, dis_ref, b_ref, w_ref, y_ref):
    d = dis_ref[...]
    hid = jnp.maximum(_cat_q(a_ref) * d + b_ref[...], 0.0)
    y = jnp.dot(hid, w_ref[...], preferred_element_type=jnp.float32) * d
    _split_q(y_ref, y)


def _tc3_body(a_ref, dis_ref, b_ref, w_ref, bl_ref, o_ref):
    d = dis_ref[...]
    hid = jnp.maximum(_cat_q(a_ref) * d + b_ref[...], 0.0)
    o_ref[...] = (
        jnp.dot(hid, w_ref[...], preferred_element_type=jnp.float32) + bl_ref[...]
    )


NQ = 2


def _tc1(x, W, degp, BR):
    N, K = x.shape
    H = W.shape[1]
    NW = degp.shape[0]
    grid = (-(-N // BR),)
    return pl.pallas_call(
        _tc1_body,
        grid=grid,
        in_specs=[
            pl.BlockSpec((BR, K), lambda i: (i, 0)),
            pl.BlockSpec((K, H), lambda i: (0, 0)),
            pl.BlockSpec((NW, BR), lambda i: (0, i)),
        ],
        out_specs=[
            pl.BlockSpec((NQ, BR, H // NQ), lambda i: (0, i, 0)),
            pl.BlockSpec((BR, 1), lambda i: (i, 0)),
        ],
        out_shape=[
            jax.ShapeDtypeStruct((NQ, N, H // NQ), jnp.float32),
            jax.ShapeDtypeStruct((N, 1), jnp.float32),
        ],
    )(x, W, degp)


def _tc2(acc, dis_col, b, W, BR):
    _, N, Hq = acc.shape
    H = W.shape[1]
    grid = (-(-N // BR),)
    return pl.pallas_call(
        _tc2_body,
        grid=grid,
        in_specs=[
            pl.BlockSpec((NQ, BR, Hq), lambda i: (0, i, 0)),
            pl.BlockSpec((BR, 1), lambda i: (i, 0)),
            pl.BlockSpec((1, NQ * Hq), lambda i: (0, 0)),
            pl.BlockSpec((NQ * Hq, H), lambda i: (0, 0)),
        ],
        out_specs=pl.BlockSpec((NQ, BR, H // NQ), lambda i: (0, i, 0)),
        out_shape=jax.ShapeDtypeStruct((NQ, N, H // NQ), jnp.float32),
    )(acc, dis_col, b, W)


def _tc3(acc, dis_col, b, W, bl, BR):
    _, N, Hq = acc.shape
    DO = W.shape[1]
    grid = (-(-N // BR),)
    return pl.pallas_call(
        _tc3_body,
        grid=grid,
        in_specs=[
            pl.BlockSpec((NQ, BR, Hq), lambda i: (0, i, 0)),
            pl.BlockSpec((BR, 1), lambda i: (i, 0)),
            pl.BlockSpec((1, NQ * Hq), lambda i: (0, 0)),
            pl.BlockSpec((NQ * Hq, DO), lambda i: (0, 0)),
            pl.BlockSpec((1, DO), lambda i: (0, 0)),
        ],
        out_specs=pl.BlockSpec((BR, DO), lambda i: (i, 0)),
        out_shape=jax.ShapeDtypeStruct((N, DO), jnp.float32),
    )(acc, dis_col, b, W, bl)


def kernel(x, edge_index, edge_weight, W1, b1, W2, b2, Wl, bl):
    N = x.shape[0]
    E = edge_weight.shape[0]
    BR = 1024

    row = edge_index[0]
    col = edge_index[1]

    deg_fn, NP = _make_deg(E, N)
    degp = deg_fn(col, edge_weight).reshape(NC * NS, NP)

    QD = 256 // NQ
    edge_fn = _make_edge(E, N, QD)

    y1, dis_col = _tc1(x, W1, degp, BR)
    a1 = edge_fn(y1.reshape(NQ * N, QD), row, col, edge_weight)
    y2 = _tc2(a1.reshape(NQ, N, QD), dis_col, b1.reshape(1, -1), W2, BR)
    a2 = edge_fn(y2.reshape(NQ * N, QD), row, col, edge_weight)
    out = _tc3(a2.reshape(NQ, N, QD), dis_col, b2.reshape(1, -1), Wl,
               bl.reshape(1, -1), BR)
    return out


# speedup vs baseline: 20.8521x; 1.0202x over previous
import functools

import jax
import jax.numpy as jnp
from jax import lax
from jax.experimental import pallas as pl
from jax.experimental.pallas import tpu as pltpu
from jax.experimental.pallas import tpu_sc as plsc

NC, NS, L = 2, 16, 16


def _mesh():
    return plsc.VectorSubcoreMesh(
        core_axis_name="c", subcore_axis_name="s", num_cores=NC, num_subcores=NS
    )


def _make_deg(E, N):
    NW = NC * NS
    TPW = E // NW
    assert TPW % L == 0
    NP = (-(-N // 1024)) * 1024

    @functools.partial(
        pl.kernel,
        out_type=jax.ShapeDtypeStruct((NW * NP,), jnp.float32),
        mesh=_mesh(),
        scratch_types=[
            pltpu.VMEM((TPW,), jnp.int32),
            pltpu.VMEM((TPW,), jnp.float32),
            pltpu.VMEM((NP,), jnp.float32),
        ],
        compiler_params=pltpu.CompilerParams(needs_layout_passes=False),
    )
    def deg_kernel(col_hbm, ew_hbm, out_hbm, col_v, ew_v, deg_v):
        cid = lax.axis_index("c")
        sid = lax.axis_index("s")
        w = sid * NC + cid
        zero16 = jnp.zeros((L,), jnp.float32)
        pltpu.sync_copy(col_hbm.at[pl.ds(w * TPW, TPW)], col_v)
        pltpu.sync_copy(ew_hbm.at[pl.ds(w * TPW, TPW)], ew_v)

        def zbody(i, c):
            deg_v[pl.ds(i * L, L)] = zero16
            return c
        lax.fori_loop(0, NP // L, zbody, 0)

        def ebody(g, c):
            sl = pl.ds(g * L, L)
            plsc.addupdate_scatter(deg_v, [col_v[sl]], ew_v[sl])
            return c
        lax.fori_loop(0, TPW // L, ebody, 0)

        pltpu.sync_copy(deg_v, out_hbm.at[pl.ds(w * NP, NP)])

    return deg_kernel, NP


def _make_edge(E, N, D):
    CH = 112
    NB = 3
    TPE = E // NS
    NCH = TPE // CH
    REM = TPE - NCH * CH
    NCHP = (NCH // NB) * NB
    assert CH % L == 0 and REM % L == 0
    nfull = N // 128
    rem = N - nfull * 128
    nhop = -(-(nfull + (1 if rem else 0)) // NS)

    @functools.partial(
        pl.kernel,
        out_type=jax.ShapeDtypeStruct((NC * N, D), jnp.float32),
        mesh=_mesh(),
        scratch_types=(
            [pltpu.VMEM((CH,), jnp.int32) for _ in range(NB)]
            + [pltpu.VMEM((CH,), jnp.int32) for _ in range(NB)]
            + [pltpu.VMEM((CH,), jnp.float32) for _ in range(NB)]
            + [pltpu.VMEM((CH, D), jnp.float32) for _ in range(NB)]
            + [pltpu.VMEM((max(REM, 8),), jnp.int32)]
            + [pltpu.VMEM_SHARED((N, D), jnp.float32)]
            + [pltpu.SemaphoreType.DMA for _ in range(4 * NB)]
        ),
    )
    def edge_kernel(y_hbm, row_hbm, col_hbm, ew_hbm, out_hbm, *refs):
        idx_v = refs[0:NB]
        col_v = refs[NB:2 * NB]
        ew_v = refs[2 * NB:3 * NB]
        rows_v = refs[3 * NB:4 * NB]
        col_rem = refs[4 * NB]
        acc_sh = refs[4 * NB + 1]
        i_sem = refs[4 * NB + 2:5 * NB + 2]
        c_sem = refs[5 * NB + 2:6 * NB + 2]
        g_sem = refs[6 * NB + 2:7 * NB + 2]
        s_sem = refs[7 * NB + 2:8 * NB + 2]

        cid = lax.axis_index("c")
        sid = lax.axis_index("s")
        ebase = sid * TPE
        offv = cid * N

        def issue_idx(kk, b, first):
            if not first:
                @pl.when(kk >= NB)
                def _():
                    pltpu.make_async_copy(
                        rows_v[b], acc_sh.at[col_v[b]], s_sem[b]).wait()
            base = ebase + kk * CH
            pltpu.async_copy(row_hbm.at[pl.ds(base, CH)], idx_v[b], i_sem[b])
            pltpu.async_copy(col_hbm.at[pl.ds(base, CH)], col_v[b], c_sem[b])
            pltpu.async_copy(ew_hbm.at[pl.ds(base, CH)], ew_v[b], c_sem[b])

        def prep(kk, b):
            pltpu.make_async_copy(
                row_hbm.at[pl.ds(ebase + kk * CH, CH)], idx_v[b],
                i_sem[b]).wait()
            for g in range(CH // L):
                sl = pl.ds(g * L, L)
                idx_v[b][sl] = idx_v[b][sl] + offv
            pltpu.async_copy(y_hbm.at[idx_v[b]], rows_v[b], g_sem[b])

        def scale(rbuf, ebuf, nedges):
            def sbody(jg, c2):
                wv = ebuf[pl.ds(jg * L, L)]
                for l in range(L):
                    s = wv[l]
                    j = jg * L + l
                    for k in range(D // L):
                        sl = pl.ds(k * L, L)
                        rbuf[j, sl] = rbuf[j, sl] * s
                return c2
            lax.fori_loop(0, nedges // L, sbody, 0)

        def process(k, b):
            pltpu.make_async_copy(
                col_hbm.at[pl.ds(ebase + k * CH, CH)], col_v[b],
                c_sem[b]).wait()
            pltpu.make_async_copy(
                ew_hbm.at[pl.ds(ebase + k * CH, CH)], ew_v[b],
                c_sem[b]).wait()
            pltpu.make_async_copy(y_hbm.at[idx_v[b]], rows_v[b],
                                  g_sem[b]).wait()
            scale(rows_v[b], ew_v[b], CH)
            pltpu.async_copy(rows_v[b], acc_sh.at[col_v[b]], s_sem[b],
                             add=True)

        def ibody(ih, c):
            h = sid + ih * NS

            @pl.when(h < nfull)
            def _():
                r0 = h * 128
                pltpu.sync_copy(y_hbm.at[pl.ds(cid * N + r0, 128)],
                                acc_sh.at[pl.ds(r0, 128)])
            if rem:
                @pl.when(h == nfull)
                def _():
                    r0 = nfull * 128
                    pltpu.sync_copy(y_hbm.at[pl.ds(cid * N + r0, rem)],
                                    acc_sh.at[pl.ds(r0, rem)])
            return c
        lax.fori_loop(0, nhop, ibody, 0)
        plsc.subcore_barrier()

        issue_idx(0, 0, first=True)
        issue_idx(1, 1, first=True)
        prep(0, 0)

        def mbody(it, c):
            for b in range(NB):
                k = it * NB + b

                @pl.when(k + 1 < NCHP)
                def _():
                    prep(k + 1, (b + 1) % NB)
                process(k, b)

                @pl.when(k + 2 < NCHP)
                def _():
                    issue_idx(k + 2, (b + 2) % NB, first=False)
            return c
        lax.fori_loop(0, NCHP // NB, mbody, 0)

        for b in range(NB):
            pltpu.make_async_copy(rows_v[b], acc_sh.at[col_v[b]],
                                  s_sem[b]).wait()

        for kx in range(NCHP, NCH):
            base = ebase + kx * CH
            pltpu.sync_copy(row_hbm.at[pl.ds(base, CH)], idx_v[0])
            pltpu.sync_copy(col_hbm.at[pl.ds(base, CH)], col_v[0])
            pltpu.sync_copy(ew_hbm.at[pl.ds(base, CH)], ew_v[0])
            for g in range(CH // L):
                sl = pl.ds(g * L, L)
                idx_v[0][sl] = idx_v[0][sl] + offv
            pltpu.async_copy(y_hbm.at[idx_v[0]], rows_v[0], g_sem[0])
            pltpu.make_async_copy(y_hbm.at[idx_v[0]], rows_v[0],
                                  g_sem[0]).wait()
            scale(rows_v[0], ew_v[0], CH)
            pltpu.async_copy(rows_v[0], acc_sh.at[col_v[0]], s_sem[0],
                             add=True)
            pltpu.make_async_copy(rows_v[0], acc_sh.at[col_v[0]],
                                  s_sem[0]).wait()

        if REM:
            base = ebase + NCH * CH
            pltpu.sync_copy(row_hbm.at[pl.ds(base, REM)],
                            idx_v[0].at[pl.ds(0, REM)])
            pltpu.sync_copy(col_hbm.at[pl.ds(base, REM)], col_rem)
            pltpu.sync_copy(ew_hbm.at[pl.ds(base, REM)],
                            ew_v[0].at[pl.ds(0, REM)])
            for g in range(REM // L):
                sl = pl.ds(g * L, L)
                idx_v[0][sl] = idx_v[0][sl] + offv
            pltpu.async_copy(y_hbm.at[idx_v[0].at[pl.ds(0, REM)]],
                             rows_v[0].at[pl.ds(0, REM)], g_sem[0])
            pltpu.make_async_copy(y_hbm.at[idx_v[0].at[pl.ds(0, REM)]],
                                  rows_v[0].at[pl.ds(0, REM)], g_sem[0]).wait()
            scale(rows_v[0], ew_v[0], REM)
            pltpu.async_copy(rows_v[0].at[pl.ds(0, REM)],
                             acc_sh.at[col_rem], s_sem[0], add=True)
            pltpu.make_async_copy(rows_v[0].at[pl.ds(0, REM)],
                                  acc_sh.at[col_rem], s_sem[0]).wait()

        plsc.subcore_barrier()

        def obody(ih, c):
            h = sid + ih * NS

            @pl.when(h < nfull)
            def _():
                r0 = h * 128
                pltpu.sync_copy(acc_sh.at[pl.ds(r0, 128)],
                                out_hbm.at[pl.ds(cid * N + r0, 128)])
            if rem:
                @pl.when(h == nfull)
                def _():
                    r0 = nfull * 128
                    pltpu.sync_copy(acc_sh.at[pl.ds(r0, rem)],
                                    out_hbm.at[pl.ds(cid * N + r0, rem)])
            return c
        lax.fori_loop(0, nhop, obody, 0)

    return edge_kernel


def _split_q(y_ref, y):
    nq = y_ref.shape[0]
    qd = y.shape[1] // nq
    for q in range(nq):
        y_ref[q] = y[:, q * qd:(q + 1) * qd]


def _cat_q(a_ref):
    return jnp.concatenate([a_ref[q] for q in range(a_ref.shape[0])], axis=1)


def _tc1_body(x_ref, w_ref, degp_ref, y_ref, dis_ref):
    dsum = jnp.sum(degp_ref[...], axis=0) + 1.0
    dis = jnp.where(dsum > 0, lax.rsqrt(dsum), 0.0)[:, None]
    xw = jnp.dot(x_ref[...], w_ref[...], preferred_element_type=jnp.float32)
    _split_q(y_ref, xw * dis)
    dis_ref[...] = dis


def _tc2_body(a_ref, dis_ref, b_ref, w_ref, y_ref):
    d = dis_ref[...]
    hid = jnp.maximum(_cat_q(a_ref) * d + b_ref[...], 0.0)
    y = jnp.dot(hid, w_ref[...], preferred_element_type=jnp.float32) * d
    _split_q(y_ref, y)


def _tc3_body(a_ref, dis_ref, b_ref, w_ref, bl_ref, o_ref):
    d = dis_ref[...]
    hid = jnp.maximum(_cat_q(a_ref) * d + b_ref[...], 0.0)
    o_ref[...] = (
        jnp.dot(hid, w_ref[...], preferred_element_type=jnp.float32) + bl_ref[...]
    )


NQ = 2


def _tc1(x, W, degp, BR):
    N, K = x.shape
    H = W.shape[1]
    NW = degp.shape[0]
    grid = (-(-N // BR),)
    return pl.pallas_call(
        _tc1_body,
        grid=grid,
        in_specs=[
            pl.BlockSpec((BR, K), lambda i: (i, 0)),
            pl.BlockSpec((K, H), lambda i: (0, 0)),
            pl.BlockSpec((NW, BR), lambda i: (0, i)),
        ],
        out_specs=[
            pl.BlockSpec((NQ, BR, H // NQ), lambda i: (0, i, 0)),
            pl.BlockSpec((BR, 1), lambda i: (i, 0)),
        ],
        out_shape=[
            jax.ShapeDtypeStruct((NQ, N, H // NQ), jnp.float32),
            jax.ShapeDtypeStruct((N, 1), jnp.float32),
        ],
    )(x, W, degp)


def _tc2(acc, dis_col, b, W, BR):
    _, N, Hq = acc.shape
    H = W.shape[1]
    grid = (-(-N // BR),)
    return pl.pallas_call(
        _tc2_body,
        grid=grid,
        in_specs=[
            pl.BlockSpec((NQ, BR, Hq), lambda i: (0, i, 0)),
            pl.BlockSpec((BR, 1), lambda i: (i, 0)),
            pl.BlockSpec((1, NQ * Hq), lambda i: (0, 0)),
            pl.BlockSpec((NQ * Hq, H), lambda i: (0, 0)),
        ],
        out_specs=pl.BlockSpec((NQ, BR, H // NQ), lambda i: (0, i, 0)),
        out_shape=jax.ShapeDtypeStruct((NQ, N, H // NQ), jnp.float32),
    )(acc, dis_col, b, W)


def _tc3(acc, dis_col, b, W, bl, BR):
    _, N, Hq = acc.shape
    DO = W.shape[1]
    grid = (-(-N // BR),)
    return pl.pallas_call(
        _tc3_body,
        grid=grid,
        in_specs=[
            pl.BlockSpec((NQ, BR, Hq), lambda i: (0, i, 0)),
            pl.BlockSpec((BR, 1), lambda i: (i, 0)),
            pl.BlockSpec((1, NQ * Hq), lambda i: (0, 0)),
            pl.BlockSpec((NQ * Hq, DO), lambda i: (0, 0)),
            pl.BlockSpec((1, DO), lambda i: (0, 0)),
        ],
        out_specs=pl.BlockSpec((BR, DO), lambda i: (i, 0)),
        out_shape=jax.ShapeDtypeStruct((N, DO), jnp.float32),
    )(acc, dis_col, b, W, bl)


def kernel(x, edge_index, edge_weight, W1, b1, W2, b2, Wl, bl):
    N = x.shape[0]
    E = edge_weight.shape[0]
    BR = 1024

    row = edge_index[0]
    col = edge_index[1]

    deg_fn, NP = _make_deg(E, N)
    degp = deg_fn(col, edge_weight).reshape(NC * NS, NP)

    QD = 256 // NQ
    edge_fn = _make_edge(E, N, QD)

    y1, dis_col = _tc1(x, W1, degp, BR)
    a1 = edge_fn(y1.reshape(NQ * N, QD), row, col, edge_weight)
    y2 = _tc2(a1.reshape(NQ, N, QD), dis_col, b1.reshape(1, -1), W2, BR)
    a2 = edge_fn(y2.reshape(NQ * N, QD), row, col, edge_weight)
    out = _tc3(a2.reshape(NQ, N, QD), dis_col, b2.reshape(1, -1), Wl,
               bl.reshape(1, -1), BR)
    return out


# speedup vs baseline: 21.4742x; 1.0298x over previous
import functools

import jax
import jax.numpy as jnp
from jax import lax
from jax.experimental import pallas as pl
from jax.experimental.pallas import tpu as pltpu
from jax.experimental.pallas import tpu_sc as plsc

NC, NS, L = 2, 16, 16


def _mesh():
    return plsc.VectorSubcoreMesh(
        core_axis_name="c", subcore_axis_name="s", num_cores=NC, num_subcores=NS
    )


def _make_deg(E, N):
    NW = NC * NS
    TPW = E // NW
    assert TPW % L == 0
    NP = (-(-N // 1024)) * 1024

    @functools.partial(
        pl.kernel,
        out_type=jax.ShapeDtypeStruct((NW * NP,), jnp.float32),
        mesh=_mesh(),
        scratch_types=[
            pltpu.VMEM((TPW,), jnp.int32),
            pltpu.VMEM((TPW,), jnp.float32),
            pltpu.VMEM((NP,), jnp.float32),
        ],
        compiler_params=pltpu.CompilerParams(needs_layout_passes=False),
    )
    def deg_kernel(col_hbm, ew_hbm, out_hbm, col_v, ew_v, deg_v):
        cid = lax.axis_index("c")
        sid = lax.axis_index("s")
        w = sid * NC + cid
        zero16 = jnp.zeros((L,), jnp.float32)
        pltpu.sync_copy(col_hbm.at[pl.ds(w * TPW, TPW)], col_v)
        pltpu.sync_copy(ew_hbm.at[pl.ds(w * TPW, TPW)], ew_v)

        def zbody(i, c):
            deg_v[pl.ds(i * L, L)] = zero16
            return c
        lax.fori_loop(0, NP // L, zbody, 0)

        def ebody(g, c):
            sl = pl.ds(g * L, L)
            plsc.addupdate_scatter(deg_v, [col_v[sl]], ew_v[sl])
            return c
        lax.fori_loop(0, TPW // L, ebody, 0)

        pltpu.sync_copy(deg_v, out_hbm.at[pl.ds(w * NP, NP)])

    return deg_kernel, NP


def _make_edge(E, N, D):
    CH = 128
    NB = 3
    TPE = E // NS
    NCH = TPE // CH
    REM = TPE - NCH * CH
    NCHP = (NCH // NB) * NB
    assert CH % L == 0 and REM % L == 0
    nfull = N // 128
    rem = N - nfull * 128
    nhop = -(-(nfull + (1 if rem else 0)) // NS)

    @functools.partial(
        pl.kernel,
        out_type=jax.ShapeDtypeStruct((NC * N, D), jnp.float32),
        mesh=_mesh(),
        scratch_types=(
            [pltpu.VMEM((CH,), jnp.int32) for _ in range(NB)]
            + [pltpu.VMEM((CH,), jnp.int32) for _ in range(NB)]
            + [pltpu.VMEM((CH,), jnp.float32) for _ in range(NB)]
            + [pltpu.VMEM((CH, D), jnp.float32) for _ in range(NB)]
            + [pltpu.VMEM((max(REM, 8),), jnp.int32)]
            + [pltpu.VMEM_SHARED((N, D), jnp.float32)]
            + [pltpu.SemaphoreType.DMA for _ in range(4 * NB)]
        ),
    )
    def edge_kernel(y_hbm, row_hbm, col_hbm, ew_hbm, out_hbm, *refs):
        idx_v = refs[0:NB]
        col_v = refs[NB:2 * NB]
        ew_v = refs[2 * NB:3 * NB]
        rows_v = refs[3 * NB:4 * NB]
        col_rem = refs[4 * NB]
        acc_sh = refs[4 * NB + 1]
        i_sem = refs[4 * NB + 2:5 * NB + 2]
        c_sem = refs[5 * NB + 2:6 * NB + 2]
        g_sem = refs[6 * NB + 2:7 * NB + 2]
        s_sem = refs[7 * NB + 2:8 * NB + 2]

        cid = lax.axis_index("c")
        sid = lax.axis_index("s")
        ebase = sid * TPE
        offv = cid * N

        def issue_idx(kk, b, first):
            if not first:
                @pl.when(kk >= NB)
                def _():
                    pltpu.make_async_copy(
                        rows_v[b], acc_sh.at[col_v[b]], s_sem[b]).wait()
            base = ebase + kk * CH
            pltpu.async_copy(row_hbm.at[pl.ds(base, CH)], idx_v[b], i_sem[b])
            pltpu.async_copy(col_hbm.at[pl.ds(base, CH)], col_v[b], c_sem[b])
            pltpu.async_copy(ew_hbm.at[pl.ds(base, CH)], ew_v[b], c_sem[b])

        def prep(kk, b):
            pltpu.make_async_copy(
                row_hbm.at[pl.ds(ebase + kk * CH, CH)], idx_v[b],
                i_sem[b]).wait()
            for g in range(CH // L):
                sl = pl.ds(g * L, L)
                idx_v[b][sl] = idx_v[b][sl] + offv
            pltpu.async_copy(y_hbm.at[idx_v[b]], rows_v[b], g_sem[b])

        def scale(rbuf, ebuf, nedges):
            def sbody(jg, c2):
                wv = ebuf[pl.ds(jg * L, L)]
                for l in range(L):
                    s = wv[l]
                    j = jg * L + l
                    for k in range(D // L):
                        sl = pl.ds(k * L, L)
                        rbuf[j, sl] = rbuf[j, sl] * s
                return c2
            lax.fori_loop(0, nedges // L, sbody, 0)

        def process(k, b):
            pltpu.make_async_copy(
                col_hbm.at[pl.ds(ebase + k * CH, CH)], col_v[b],
                c_sem[b]).wait()
            pltpu.make_async_copy(
                ew_hbm.at[pl.ds(ebase + k * CH, CH)], ew_v[b],
                c_sem[b]).wait()
            pltpu.make_async_copy(y_hbm.at[idx_v[b]], rows_v[b],
                                  g_sem[b]).wait()
            scale(rows_v[b], ew_v[b], CH)
            pltpu.async_copy(rows_v[b], acc_sh.at[col_v[b]], s_sem[b],
                             add=True)

        def ibody(ih, c):
            h = sid + ih * NS

            @pl.when(h < nfull)
            def _():
                r0 = h * 128
                pltpu.sync_copy(y_hbm.at[pl.ds(cid * N + r0, 128)],
                                acc_sh.at[pl.ds(r0, 128)])
            if rem:
                @pl.when(h == nfull)
                def _():
                    r0 = nfull * 128
                    pltpu.sync_copy(y_hbm.at[pl.ds(cid * N + r0, rem)],
                                    acc_sh.at[pl.ds(r0, rem)])
            return c
        lax.fori_loop(0, nhop, ibody, 0)
        plsc.subcore_barrier()

        issue_idx(0, 0, first=True)
        issue_idx(1, 1, first=True)
        prep(0, 0)

        def mbody(it, c):
            for b in range(NB):
                k = it * NB + b

                @pl.when(k + 1 < NCHP)
                def _():
                    prep(k + 1, (b + 1) % NB)
                process(k, b)

                @pl.when(k + 2 < NCHP)
                def _():
                    issue_idx(k + 2, (b + 2) % NB, first=False)
            return c
        lax.fori_loop(0, NCHP // NB, mbody, 0)

        for b in range(NB):
            pltpu.make_async_copy(rows_v[b], acc_sh.at[col_v[b]],
                                  s_sem[b]).wait()

        for kx in range(NCHP, NCH):
            base = ebase + kx * CH
            pltpu.sync_copy(row_hbm.at[pl.ds(base, CH)], idx_v[0])
            pltpu.sync_copy(col_hbm.at[pl.ds(base, CH)], col_v[0])
            pltpu.sync_copy(ew_hbm.at[pl.ds(base, CH)], ew_v[0])
            for g in range(CH // L):
                sl = pl.ds(g * L, L)
                idx_v[0][sl] = idx_v[0][sl] + offv
            pltpu.async_copy(y_hbm.at[idx_v[0]], rows_v[0], g_sem[0])
            pltpu.make_async_copy(y_hbm.at[idx_v[0]], rows_v[0],
                                  g_sem[0]).wait()
            scale(rows_v[0], ew_v[0], CH)
            pltpu.async_copy(rows_v[0], acc_sh.at[col_v[0]], s_sem[0],
                             add=True)
            pltpu.make_async_copy(rows_v[0], acc_sh.at[col_v[0]],
                                  s_sem[0]).wait()

        if REM:
            base = ebase + NCH * CH
            pltpu.sync_copy(row_hbm.at[pl.ds(base, REM)],
                            idx_v[0].at[pl.ds(0, REM)])
            pltpu.sync_copy(col_hbm.at[pl.ds(base, REM)], col_rem)
            pltpu.sync_copy(ew_hbm.at[pl.ds(base, REM)],
                            ew_v[0].at[pl.ds(0, REM)])
            for g in range(REM // L):
                sl = pl.ds(g * L, L)
                idx_v[0][sl] = idx_v[0][sl] + offv
            pltpu.async_copy(y_hbm.at[idx_v[0].at[pl.ds(0, REM)]],
                             rows_v[0].at[pl.ds(0, REM)], g_sem[0])
            pltpu.make_async_copy(y_hbm.at[idx_v[0].at[pl.ds(0, REM)]],
                                  rows_v[0].at[pl.ds(0, REM)], g_sem[0]).wait()
            scale(rows_v[0], ew_v[0], REM)
            pltpu.async_copy(rows_v[0].at[pl.ds(0, REM)],
                             acc_sh.at[col_rem], s_sem[0], add=True)
            pltpu.make_async_copy(rows_v[0].at[pl.ds(0, REM)],
                                  acc_sh.at[col_rem], s_sem[0]).wait()

        plsc.subcore_barrier()

        def obody(ih, c):
            h = sid + ih * NS

            @pl.when(h < nfull)
            def _():
                r0 = h * 128
                pltpu.sync_copy(acc_sh.at[pl.ds(r0, 128)],
                                out_hbm.at[pl.ds(cid * N + r0, 128)])
            if rem:
                @pl.when(h == nfull)
                def _():
                    r0 = nfull * 128
                    pltpu.sync_copy(acc_sh.at[pl.ds(r0, rem)],
                                    out_hbm.at[pl.ds(cid * N + r0, rem)])
            return c
        lax.fori_loop(0, nhop, obody, 0)

    return edge_kernel


def _split_q(y_ref, y):
    nq = y_ref.shape[0]
    qd = y.shape[1] // nq
    for q in range(nq):
        y_ref[q] = y[:, q * qd:(q + 1) * qd]


def _cat_q(a_ref):
    return jnp.concatenate([a_ref[q] for q in range(a_ref.shape[0])], axis=1)


def _tc1_body(x_ref, w_ref, degp_ref, y_ref, dis_ref):
    dsum = jnp.sum(degp_ref[...], axis=0) + 1.0
    dis = jnp.where(dsum > 0, lax.rsqrt(dsum), 0.0)[:, None]
    xw = jnp.dot(x_ref[...], w_ref[...], preferred_element_type=jnp.float32)
    _split_q(y_ref, xw * dis)
    dis_ref[...] = dis


def _tc2_body(a_ref, dis_ref, b_ref, w_ref, y_ref):
    d = dis_ref[...]
    hid = jnp.maximum(_cat_q(a_ref) * d + b_ref[...], 0.0)
    y = jnp.dot(hid, w_ref[...], preferred_element_type=jnp.float32) * d
    _split_q(y_ref, y)


def _tc3_body(a_ref, dis_ref, b_ref, w_ref, bl_ref, o_ref):
    d = dis_ref[...]
    hid = jnp.maximum(_cat_q(a_ref) * d + b_ref[...], 0.0)
    o_ref[...] = (
        jnp.dot(hid, w_ref[...], preferred_element_type=jnp.float32) + bl_ref[...]
    )


NQ = 2


def _tc1(x, W, degp, BR):
    N, K = x.shape
    H = W.shape[1]
    NW = degp.shape[0]
    grid = (-(-N // BR),)
    return pl.pallas_call(
        _tc1_body,
        grid=grid,
        in_specs=[
            pl.BlockSpec((BR, K), lambda i: (i, 0)),
            pl.BlockSpec((K, H), lambda i: (0, 0)),
            pl.BlockSpec((NW, BR), lambda i: (0, i)),
        ],
        out_specs=[
            pl.BlockSpec((NQ, BR, H // NQ), lambda i: (0, i, 0)),
            pl.BlockSpec((BR, 1), lambda i: (i, 0)),
        ],
        out_shape=[
            jax.ShapeDtypeStruct((NQ, N, H // NQ), jnp.float32),
            jax.ShapeDtypeStruct((N, 1), jnp.float32),
        ],
    )(x, W, degp)


def _tc2(acc, dis_col, b, W, BR):
    _, N, Hq = acc.shape
    H = W.shape[1]
    grid = (-(-N // BR),)
    return pl.pallas_call(
        _tc2_body,
        grid=grid,
        in_specs=[
            pl.BlockSpec((NQ, BR, Hq), lambda i: (0, i, 0)),
            pl.BlockSpec((BR, 1), lambda i: (i, 0)),
            pl.BlockSpec((1, NQ * Hq), lambda i: (0, 0)),
            pl.BlockSpec((NQ * Hq, H), lambda i: (0, 0)),
        ],
        out_specs=pl.BlockSpec((NQ, BR, H // NQ), lambda i: (0, i, 0)),
        out_shape=jax.ShapeDtypeStruct((NQ, N, H // NQ), jnp.float32),
    )(acc, dis_col, b, W)


def _tc3(acc, dis_col, b, W, bl, BR):
    _, N, Hq = acc.shape
    DO = W.shape[1]
    grid = (-(-N // BR),)
    return pl.pallas_call(
        _tc3_body,
        grid=grid,
        in_specs=[
            pl.BlockSpec((NQ, BR, Hq), lambda i: (0, i, 0)),
            pl.BlockSpec((BR, 1), lambda i: (i, 0)),
            pl.BlockSpec((1, NQ * Hq), lambda i: (0, 0)),
            pl.BlockSpec((NQ * Hq, DO), lambda i: (0, 0)),
            pl.BlockSpec((1, DO), lambda i: (0, 0)),
        ],
        out_specs=pl.BlockSpec((BR, DO), lambda i: (i, 0)),
        out_shape=jax.ShapeDtypeStruct((N, DO), jnp.float32),
    )(acc, dis_col, b, W, bl)


def kernel(x, edge_index, edge_weight, W1, b1, W2, b2, Wl, bl):
    N = x.shape[0]
    E = edge_weight.shape[0]
    BR = 1024

    row = edge_index[0]
    col = edge_index[1]

    deg_fn, NP = _make_deg(E, N)
    degp = deg_fn(col, edge_weight).reshape(NC * NS, NP)

    QD = 256 // NQ
    edge_fn = _make_edge(E, N, QD)

    y1, dis_col = _tc1(x, W1, degp, BR)
    a1 = edge_fn(y1.reshape(NQ * N, QD), row, col, edge_weight)
    y2 = _tc2(a1.reshape(NQ, N, QD), dis_col, b1.reshape(1, -1), W2, BR)
    a2 = edge_fn(y2.reshape(NQ * N, QD), row, col, edge_weight)
    out = _tc3(a2.reshape(NQ, N, QD), dis_col, b2.reshape(1, -1), Wl,
               bl.reshape(1, -1), BR)
    return out
